# Initial kernel scaffold; baseline (speedup 1.0000x reference)
import jax
import jax.numpy as jnp
from jax.experimental import pallas as pl


def kernel(x, edge, edge_weight, param, p2s, p3s, W, b):
    raise NotImplementedError("write your pallas kernel here")


# speedup vs baseline: 10.7073x; 10.7073x over previous
import functools

import jax
import jax.numpy as jnp
from jax import lax
from jax.experimental import pallas as pl
from jax.experimental.pallas import tpu as pltpu
from jax.experimental.pallas import tpu_sc as plsc

N = 10000
E = 160000
B = 16
D = 256
NGCN = 3

NW = 32
CH = 128
NCH = 40
EPW = CH * NCH
EPAD = NW * EPW
NPAD = 10240
ROWS_PER_SUB = NPAD // 16

_NB = 400


def _h0_body(x_ref, p_ref, o_ref):
    xb = x_ref[...]
    p = jnp.abs(p_ref[...])
    acc = lax.dot_general(xb.reshape(B * _NB, D), p,
                          (((1,), (0,)), ((), ())),
                          preferred_element_type=jnp.float32)
    h = jnp.tanh(acc) * (1.0 / D)
    o_ref[...] = jnp.transpose(h.reshape(B, _NB), (1, 0))


def _h0_call(x, param):
    return pl.pallas_call(
        _h0_body,
        grid=(N // _NB,),
        in_specs=[
            pl.BlockSpec((B, _NB, D), lambda i: (0, i, 0)),
            pl.BlockSpec((D, 1), lambda i: (0, 0)),
        ],
        out_specs=pl.BlockSpec((_NB, B), lambda i: (i, 0)),
        out_shape=jax.ShapeDtypeStruct((N, B), jnp.float32),
    )(x, param)


_SC_MESH = plsc.VectorSubcoreMesh(core_axis_name="c", subcore_axis_name="s")
_SC_PARAMS = pltpu.CompilerParams(use_tc_tiling_on_sc=False)


def _sc_common(h_hbm, src_hbm, dst_hbm, srcv, dstv, gbuf, zbuf, s_sh, sem,
               s_out, c, s):
    wid = s * 2 + c
    pltpu.sync_copy(src_hbm.at[wid], srcv)
    pltpu.sync_copy(dst_hbm.at[wid], dstv)
    for r in range(16):
        zbuf[r, :] = jnp.zeros((B,), jnp.float32)
    base = s * ROWS_PER_SUB

    def zbody(k, carry):
        pltpu.sync_copy(zbuf, s_sh.at[pl.ds(base + k * 16, 16)])
        return carry

    lax.fori_loop(0, ROWS_PER_SUB // 16, zbody, 0)
    plsc.subcore_barrier()

    def ebody(j, carry):
        pltpu.async_copy(h_hbm.at[srcv.at[j]], gbuf, sem).wait()
        pltpu.sync_copy(gbuf, s_sh.at[dstv.at[j]], add=True)
        return carry

    lax.fori_loop(0, NCH, ebody, 0)
    plsc.subcore_barrier()
    pltpu.sync_copy(s_sh.at[pl.ds(base, ROWS_PER_SUB)],
                    s_out.at[pl.ds(c * NPAD + base, ROWS_PER_SUB)])


@functools.partial(
    pl.kernel,
    mesh=_SC_MESH,
    compiler_params=_SC_PARAMS,
    out_type=jax.ShapeDtypeStruct((2 * NPAD, B), jnp.float32),
    scratch_types=[
        pltpu.VMEM((NCH, CH), jnp.int32),
        pltpu.VMEM((NCH, CH), jnp.int32),
        pltpu.VMEM((CH, B), jnp.float32),
        pltpu.VMEM((16, B), jnp.float32),
        pltpu.VMEM_SHARED((NPAD, B), jnp.float32),
        pltpu.SemaphoreType.DMA,
    ],
)
def _sc_layer(h_hbm, src_hbm, dst_hbm, s_out, srcv, dstv, gbuf, zbuf, s_sh,
              sem):
    c = lax.axis_index("c")
    s = lax.axis_index("s")
    _sc_common(h_hbm, src_hbm, dst_hbm, srcv, dstv, gbuf, zbuf, s_sh, sem,
               s_out, c, s)


@functools.partial(
    pl.kernel,
    mesh=_SC_MESH,
    compiler_params=_SC_PARAMS,
    out_type=(jax.ShapeDtypeStruct((2 * NPAD, B), jnp.float32),
              jax.ShapeDtypeStruct((2 * NPAD, B), jnp.float32)),
    scratch_types=[
        pltpu.VMEM((NCH, CH), jnp.int32),
        pltpu.VMEM((NCH, CH), jnp.int32),
        pltpu.VMEM((CH, B), jnp.float32),
        pltpu.VMEM((16, B), jnp.float32),
        pltpu.VMEM((CH, B), jnp.float32),
        pltpu.VMEM_SHARED((NPAD, B), jnp.float32),
        pltpu.VMEM_SHARED((NPAD, B), jnp.float32),
        pltpu.SemaphoreType.DMA,
    ],
)
def _sc_layer_cnt(h_hbm, src_hbm, dst_hbm, ones_hbm, s_out, c_out, srcv, dstv,
                  gbuf, zbuf, obuf, s_sh, c_sh, sem):
    c = lax.axis_index("c")
    s = lax.axis_index("s")
    wid = s * 2 + c
    pltpu.sync_copy(src_hbm.at[wid], srcv)
    pltpu.sync_copy(dst_hbm.at[wid], dstv)
    pltpu.sync_copy(ones_hbm, obuf)
    for r in range(16):
        zbuf[r, :] = jnp.zeros((B,), jnp.float32)
    base = s * ROWS_PER_SUB

    def zbody(k, carry):
        pltpu.sync_copy(zbuf, s_sh.at[pl.ds(base + k * 16, 16)])
        pltpu.sync_copy(zbuf, c_sh.at[pl.ds(base + k * 16, 16)])
        return carry

    lax.fori_loop(0, ROWS_PER_SUB // 16, zbody, 0)
    plsc.subcore_barrier()

    def ebody(j, carry):
        pltpu.async_copy(h_hbm.at[srcv.at[j]], gbuf, sem).wait()
        pltpu.sync_copy(gbuf, s_sh.at[dstv.at[j]], add=True)
        pltpu.sync_copy(obuf, c_sh.at[dstv.at[j]], add=True)
        return carry

    lax.fori_loop(0, NCH, ebody, 0)
    plsc.subcore_barrier()
    pltpu.sync_copy(s_sh.at[pl.ds(base, ROWS_PER_SUB)],
                    s_out.at[pl.ds(c * NPAD + base, ROWS_PER_SUB)])
    pltpu.sync_copy(c_sh.at[pl.ds(base, ROWS_PER_SUB)],
                    c_out.at[pl.ds(c * NPAD + base, ROWS_PER_SUB)])


def _merge1_body(s_ref, c_ref, h_ref, p2_ref, p3_ref, hn_ref, cnt_ref):
    S = s_ref[0:NPAD, :] + s_ref[NPAD:2 * NPAD, :]
    cnt = c_ref[0:NPAD, :] + c_ref[NPAD:2 * NPAD, :]
    inv = 1.0 / jnp.maximum(cnt, 1.0)
    p2 = p2_ref[0, 0]
    p3 = p3_ref[0, 0]
    hn_ref[...] = jnp.tanh(p2 * S * inv + p3 * cnt * inv * h_ref[...])
    cnt_ref[...] = cnt


def _merge1_call(s2, c2, h, p2, p3):
    return pl.pallas_call(
        _merge1_body,
        out_shape=(jax.ShapeDtypeStruct((NPAD, B), jnp.float32),
                   jax.ShapeDtypeStruct((NPAD, B), jnp.float32)),
    )(s2, c2, h, p2, p3)


def _merge2_body(s_ref, cnt_ref, h_ref, p2_ref, p3_ref, hn_ref):
    S = s_ref[0:NPAD, :] + s_ref[NPAD:2 * NPAD, :]
    cnt = cnt_ref[...]
    inv = 1.0 / jnp.maximum(cnt, 1.0)
    p2 = p2_ref[0, 0]
    p3 = p3_ref[0, 0]
    hn_ref[...] = jnp.tanh(p2 * S * inv + p3 * cnt * inv * h_ref[...])


def _merge2_call(s2, cnt, h, p2, p3):
    return pl.pallas_call(
        _merge2_body,
        out_shape=jax.ShapeDtypeStruct((NPAD, B), jnp.float32),
    )(s2, cnt, h, p2, p3)


def _merge3_body(s_ref, cnt_ref, h_ref, p2_ref, p3_ref, w_ref, b_ref,
                 out_ref):
    S = s_ref[0:NPAD, :] + s_ref[NPAD:2 * NPAD, :]
    cnt = cnt_ref[...]
    inv = 1.0 / jnp.maximum(cnt, 1.0)
    p2 = p2_ref[0, 0]
    p3 = p3_ref[0, 0]
    h3 = jnp.tanh(p2 * S * inv + p3 * cnt * inv * h_ref[...])
    out = lax.dot_general(h3, w_ref[...], (((0,), (0,)), ((), ())),
                          preferred_element_type=jnp.float32)
    out_ref[...] = out + b_ref[0, 0]


def _merge3_call(s2, cnt, h, p2, p3, wT, b2):
    return pl.pallas_call(
        _merge3_body,
        out_shape=jax.ShapeDtypeStruct((B, 1), jnp.float32),
    )(s2, cnt, h, p2, p3, wT, b2)


def kernel(x, edge, edge_weight, param, p2s, p3s, W, b):
    h0 = _h0_call(x, param)
    h = jnp.concatenate(
        [h0, jnp.zeros((NPAD - N, B), jnp.float32)], axis=0)

    pad = EPAD - E
    srcp = jnp.concatenate(
        [edge[0], jnp.full((pad,), N, jnp.int32)]).reshape(NW, NCH, CH)
    dstp = jnp.concatenate(
        [edge[1], jnp.full((pad,), N, jnp.int32)]).reshape(NW, NCH, CH)
    ones = jnp.ones((CH, B), jnp.float32)

    a2 = jnp.abs(p2s).reshape(NGCN, 1, 1).astype(jnp.float32)
    a3 = jnp.abs(p3s).reshape(NGCN, 1, 1).astype(jnp.float32)

    s1, c1 = _sc_layer_cnt(h, srcp, dstp, ones)
    h, cnt = _merge1_call(s1, c1, h, a2[0], a3[0])
    s2 = _sc_layer(h, srcp, dstp)
    h = _merge2_call(s2, cnt, h, a2[1], a3[1])
    s3 = _sc_layer(h, srcp, dstp)

    wT = jnp.concatenate(
        [jnp.transpose(W, (1, 0)),
         jnp.zeros((NPAD - N, 1), jnp.float32)], axis=0)
    b2 = b.reshape(1, 1).astype(jnp.float32)
    return _merge3_call(s3, cnt, h, a2[2], a3[2], wT, b2)


# speedup vs baseline: 12.7322x; 1.1891x over previous
import functools

import jax
import jax.numpy as jnp
from jax import lax
from jax.experimental import pallas as pl
from jax.experimental.pallas import tpu as pltpu
from jax.experimental.pallas import tpu_sc as plsc

N = 10000
E = 160000
B = 16
D = 256
NGCN = 3

NW = 32
CH = 128
NCH = 40
EPW = CH * NCH
EPAD = NW * EPW
NPAD = 10240
ROWS_PER_SUB = NPAD // 16

_NB = 400


def _h0_body(x_ref, p_ref, o_ref):
    xb = x_ref[...]
    p = jnp.abs(p_ref[...])
    acc = lax.dot_general(xb.reshape(B * _NB, D), p,
                          (((1,), (0,)), ((), ())),
                          preferred_element_type=jnp.float32)
    h = jnp.tanh(acc) * (1.0 / D)
    o_ref[...] = jnp.transpose(h.reshape(B, _NB), (1, 0))


def _h0_call(x, param):
    return pl.pallas_call(
        _h0_body,
        grid=(N // _NB,),
        in_specs=[
            pl.BlockSpec((B, _NB, D), lambda i: (0, i, 0)),
            pl.BlockSpec((D, 1), lambda i: (0, 0)),
        ],
        out_specs=pl.BlockSpec((_NB, B), lambda i: (i, 0)),
        out_shape=jax.ShapeDtypeStruct((N, B), jnp.float32),
    )(x, param)


_SC_MESH = plsc.VectorSubcoreMesh(core_axis_name="c", subcore_axis_name="s")
_SC_PARAMS = pltpu.CompilerParams(use_tc_tiling_on_sc=False)


def _start_gather(h_hbm, srcv, j, buf, sem):
    pltpu.async_copy(h_hbm.at[srcv.at[j]], buf, sem)


def _wait_gather(h_hbm, srcv, j, buf, sem):
    pltpu.make_async_copy(h_hbm.at[srcv.at[j]], buf, sem).wait()


def _zero_shared(zbuf, s_sh, base):
    for r in range(16):
        zbuf[r, :] = jnp.zeros((B,), jnp.float32)

    def zbody(k, carry):
        pltpu.sync_copy(zbuf, s_sh.at[pl.ds(base + k * 16, 16)])
        return carry

    lax.fori_loop(0, ROWS_PER_SUB // 16, zbody, 0)


@functools.partial(
    pl.kernel,
    mesh=_SC_MESH,
    compiler_params=_SC_PARAMS,
    out_type=jax.ShapeDtypeStruct((2 * NPAD, B), jnp.float32),
    scratch_types=[
        pltpu.VMEM((NCH, CH), jnp.int32),
        pltpu.VMEM((NCH, CH), jnp.int32),
        pltpu.VMEM((CH, B), jnp.float32),
        pltpu.VMEM((CH, B), jnp.float32),
        pltpu.VMEM((16, B), jnp.float32),
        pltpu.VMEM_SHARED((NPAD, B), jnp.float32),
        pltpu.SemaphoreType.DMA,
        pltpu.SemaphoreType.DMA,
    ],
)
def _sc_layer(h_hbm, src_hbm, dst_hbm, s_out, srcv, dstv, g0, g1, zbuf, s_sh,
              sem0, sem1):
    c = lax.axis_index("c")
    s = lax.axis_index("s")
    wid = s * 2 + c
    pltpu.sync_copy(src_hbm.at[wid], srcv)
    pltpu.sync_copy(dst_hbm.at[wid], dstv)
    base = s * ROWS_PER_SUB
    _zero_shared(zbuf, s_sh, base)
    plsc.subcore_barrier()

    _start_gather(h_hbm, srcv, 0, g0, sem0)
    ngrp = NCH // 2

    def group(g, carry):
        j0 = 2 * g
        _start_gather(h_hbm, srcv, j0 + 1, g1, sem1)
        _wait_gather(h_hbm, srcv, j0, g0, sem0)
        pltpu.sync_copy(g0, s_sh.at[dstv.at[j0]], add=True)

        @pl.when(g < ngrp - 1)
        def _():
            _start_gather(h_hbm, srcv, j0 + 2, g0, sem0)

        _wait_gather(h_hbm, srcv, j0 + 1, g1, sem1)
        pltpu.sync_copy(g1, s_sh.at[dstv.at[j0 + 1]], add=True)
        return carry

    lax.fori_loop(0, ngrp, group, 0)
    plsc.subcore_barrier()
    pltpu.sync_copy(s_sh.at[pl.ds(base, ROWS_PER_SUB)],
                    s_out.at[pl.ds(c * NPAD + base, ROWS_PER_SUB)])


@functools.partial(
    pl.kernel,
    mesh=_SC_MESH,
    compiler_params=_SC_PARAMS,
    out_type=jax.ShapeDtypeStruct((2 * NPAD, B), jnp.float32),
    scratch_types=[
        pltpu.VMEM((NCH, CH), jnp.int32),
        pltpu.VMEM((CH, B), jnp.float32),
        pltpu.VMEM((16, B), jnp.float32),
        pltpu.VMEM_SHARED((NPAD, B), jnp.float32),
        pltpu.SemaphoreType.DMA,
    ],
)
def _sc_cnt(dst_hbm, ones_hbm, c_out, dstv, obuf, zbuf, c_sh, sem):
    c = lax.axis_index("c")
    s = lax.axis_index("s")
    wid = s * 2 + c
    pltpu.sync_copy(dst_hbm.at[wid], dstv)
    pltpu.sync_copy(ones_hbm, obuf)
    base = s * ROWS_PER_SUB
    _zero_shared(zbuf, c_sh, base)
    plsc.subcore_barrier()

    def ebody(j, carry):
        pltpu.sync_copy(obuf, c_sh.at[dstv.at[j]], add=True)
        return carry

    lax.fori_loop(0, NCH, ebody, 0)
    plsc.subcore_barrier()
    pltpu.sync_copy(c_sh.at[pl.ds(base, ROWS_PER_SUB)],
                    c_out.at[pl.ds(c * NPAD + base, ROWS_PER_SUB)])


def _merge1_body(s_ref, c_ref, h_ref, p2_ref, p3_ref, hn_ref, cnt_ref):
    S = s_ref[0:NPAD, :] + s_ref[NPAD:2 * NPAD, :]
    cnt = c_ref[0:NPAD, :] + c_ref[NPAD:2 * NPAD, :]
    inv = 1.0 / jnp.maximum(cnt, 1.0)
    p2 = p2_ref[0, 0]
    p3 = p3_ref[0, 0]
    hn_ref[...] = jnp.tanh(p2 * S * inv + p3 * cnt * inv * h_ref[...])
    cnt_ref[...] = cnt


def _merge1_call(s2, c2, h, p2, p3):
    return pl.pallas_call(
        _merge1_body,
        out_shape=(jax.ShapeDtypeStruct((NPAD, B), jnp.float32),
                   jax.ShapeDtypeStruct((NPAD, B), jnp.float32)),
    )(s2, c2, h, p2, p3)


def _merge2_body(s_ref, cnt_ref, h_ref, p2_ref, p3_ref, hn_ref):
    S = s_ref[0:NPAD, :] + s_ref[NPAD:2 * NPAD, :]
    cnt = cnt_ref[...]
    inv = 1.0 / jnp.maximum(cnt, 1.0)
    p2 = p2_ref[0, 0]
    p3 = p3_ref[0, 0]
    hn_ref[...] = jnp.tanh(p2 * S * inv + p3 * cnt * inv * h_ref[...])


def _merge2_call(s2, cnt, h, p2, p3):
    return pl.pallas_call(
        _merge2_body,
        out_shape=jax.ShapeDtypeStruct((NPAD, B), jnp.float32),
    )(s2, cnt, h, p2, p3)


def _merge3_body(s_ref, cnt_ref, h_ref, p2_ref, p3_ref, w_ref, b_ref,
                 out_ref):
    S = s_ref[0:NPAD, :] + s_ref[NPAD:2 * NPAD, :]
    cnt = cnt_ref[...]
    inv = 1.0 / jnp.maximum(cnt, 1.0)
    p2 = p2_ref[0, 0]
    p3 = p3_ref[0, 0]
    h3 = jnp.tanh(p2 * S * inv + p3 * cnt * inv * h_ref[...])
    out = lax.dot_general(h3, w_ref[...], (((0,), (0,)), ((), ())),
                          preferred_element_type=jnp.float32)
    out_ref[...] = out + b_ref[0, 0]


def _merge3_call(s2, cnt, h, p2, p3, wT, b2):
    return pl.pallas_call(
        _merge3_body,
        out_shape=jax.ShapeDtypeStruct((B, 1), jnp.float32),
    )(s2, cnt, h, p2, p3, wT, b2)


def kernel(x, edge, edge_weight, param, p2s, p3s, W, b):
    pad = EPAD - E
    srcp = jnp.concatenate(
        [edge[0], jnp.full((pad,), N, jnp.int32)]).reshape(NW, NCH, CH)
    dstp = jnp.concatenate(
        [edge[1], jnp.full((pad,), N, jnp.int32)]).reshape(NW, NCH, CH)
    ones = jnp.ones((CH, B), jnp.float32)

    c1 = _sc_cnt(dstp, ones)
    h0 = _h0_call(x, param)
    h = jnp.concatenate(
        [h0, jnp.zeros((NPAD - N, B), jnp.float32)], axis=0)

    a2 = jnp.abs(p2s).reshape(NGCN, 1, 1).astype(jnp.float32)
    a3 = jnp.abs(p3s).reshape(NGCN, 1, 1).astype(jnp.float32)

    s1 = _sc_layer(h, srcp, dstp)
    h, cnt = _merge1_call(s1, c1, h, a2[0], a3[0])
    s2 = _sc_layer(h, srcp, dstp)
    h = _merge2_call(s2, cnt, h, a2[1], a3[1])
    s3 = _sc_layer(h, srcp, dstp)

    wT = jnp.concatenate(
        [jnp.transpose(W, (1, 0)),
         jnp.zeros((NPAD - N, 1), jnp.float32)], axis=0)
    b2 = b.reshape(1, 1).astype(jnp.float32)
    return _merge3_call(s3, cnt, h, a2[2], a3[2], wT, b2)


# speedup vs baseline: 12.7968x; 1.0051x over previous
import functools

import jax
import jax.numpy as jnp
from jax import lax
from jax.experimental import pallas as pl
from jax.experimental.pallas import tpu as pltpu
from jax.experimental.pallas import tpu_sc as plsc

N = 10000
E = 160000
B = 16
D = 256
NGCN = 3

NW = 32
CH = 512
NCH = 10
EPW = CH * NCH
EPAD = NW * EPW
NPAD = 10240
ROWS_PER_SUB = NPAD // 16

_NB = 400


def _h0_body(x_ref, p_ref, o_ref):
    xb = x_ref[...]
    p = jnp.abs(p_ref[...])
    acc = lax.dot_general(xb.reshape(B * _NB, D), p,
                          (((1,), (0,)), ((), ())),
                          preferred_element_type=jnp.float32)
    h = jnp.tanh(acc) * (1.0 / D)
    o_ref[...] = jnp.transpose(h.reshape(B, _NB), (1, 0))


def _h0_call(x, param):
    return pl.pallas_call(
        _h0_body,
        grid=(N // _NB,),
        in_specs=[
            pl.BlockSpec((B, _NB, D), lambda i: (0, i, 0)),
            pl.BlockSpec((D, 1), lambda i: (0, 0)),
        ],
        out_specs=pl.BlockSpec((_NB, B), lambda i: (i, 0)),
        out_shape=jax.ShapeDtypeStruct((N, B), jnp.float32),
    )(x, param)


_SC_MESH = plsc.VectorSubcoreMesh(core_axis_name="c", subcore_axis_name="s")
_SC_PARAMS = pltpu.CompilerParams(use_tc_tiling_on_sc=False)


def _start_gather(h_hbm, srcv, j, buf, sem):
    pltpu.async_copy(h_hbm.at[srcv.at[j]], buf, sem)


def _wait_gather(h_hbm, srcv, j, buf, sem):
    pltpu.make_async_copy(h_hbm.at[srcv.at[j]], buf, sem).wait()


def _zero_shared(zbuf, s_sh, base):
    for r in range(16):
        zbuf[r, :] = jnp.zeros((B,), jnp.float32)

    def zbody(k, carry):
        pltpu.sync_copy(zbuf, s_sh.at[pl.ds(base + k * 16, 16)])
        return carry

    lax.fori_loop(0, ROWS_PER_SUB // 16, zbody, 0)


@functools.partial(
    pl.kernel,
    mesh=_SC_MESH,
    compiler_params=_SC_PARAMS,
    out_type=jax.ShapeDtypeStruct((2 * NPAD, B), jnp.float32),
    scratch_types=[
        pltpu.VMEM((NCH, CH), jnp.int32),
        pltpu.VMEM((NCH, CH), jnp.int32),
        pltpu.VMEM((CH, B), jnp.float32),
        pltpu.VMEM((CH, B), jnp.float32),
        pltpu.VMEM((16, B), jnp.float32),
        pltpu.VMEM_SHARED((NPAD, B), jnp.float32),
        pltpu.SemaphoreType.DMA,
        pltpu.SemaphoreType.DMA,
    ],
)
def _sc_layer(h_hbm, src_hbm, dst_hbm, s_out, srcv, dstv, g0, g1, zbuf, s_sh,
              sem0, sem1):
    c = lax.axis_index("c")
    s = lax.axis_index("s")
    wid = s * 2 + c
    pltpu.sync_copy(src_hbm.at[wid], srcv)
    pltpu.sync_copy(dst_hbm.at[wid], dstv)
    base = s * ROWS_PER_SUB
    _zero_shared(zbuf, s_sh, base)
    plsc.subcore_barrier()

    _start_gather(h_hbm, srcv, 0, g0, sem0)
    ngrp = NCH // 2

    def group(g, carry):
        j0 = 2 * g
        _start_gather(h_hbm, srcv, j0 + 1, g1, sem1)
        _wait_gather(h_hbm, srcv, j0, g0, sem0)
        pltpu.sync_copy(g0, s_sh.at[dstv.at[j0]], add=True)

        @pl.when(g < ngrp - 1)
        def _():
            _start_gather(h_hbm, srcv, j0 + 2, g0, sem0)

        _wait_gather(h_hbm, srcv, j0 + 1, g1, sem1)
        pltpu.sync_copy(g1, s_sh.at[dstv.at[j0 + 1]], add=True)
        return carry

    lax.fori_loop(0, ngrp, group, 0)
    plsc.subcore_barrier()
    pltpu.sync_copy(s_sh.at[pl.ds(base, ROWS_PER_SUB)],
                    s_out.at[pl.ds(c * NPAD + base, ROWS_PER_SUB)])


@functools.partial(
    pl.kernel,
    mesh=_SC_MESH,
    compiler_params=_SC_PARAMS,
    out_type=jax.ShapeDtypeStruct((2 * NPAD, B), jnp.float32),
    scratch_types=[
        pltpu.VMEM((NCH, CH), jnp.int32),
        pltpu.VMEM((CH, B), jnp.float32),
        pltpu.VMEM((16, B), jnp.float32),
        pltpu.VMEM_SHARED((NPAD, B), jnp.float32),
        pltpu.SemaphoreType.DMA,
    ],
)
def _sc_cnt(dst_hbm, ones_hbm, c_out, dstv, obuf, zbuf, c_sh, sem):
    c = lax.axis_index("c")
    s = lax.axis_index("s")
    wid = s * 2 + c
    pltpu.sync_copy(dst_hbm.at[wid], dstv)
    pltpu.sync_copy(ones_hbm, obuf)
    base = s * ROWS_PER_SUB
    _zero_shared(zbuf, c_sh, base)
    plsc.subcore_barrier()

    def ebody(j, carry):
        pltpu.sync_copy(obuf, c_sh.at[dstv.at[j]], add=True)
        return carry

    lax.fori_loop(0, NCH, ebody, 0)
    plsc.subcore_barrier()
    pltpu.sync_copy(c_sh.at[pl.ds(base, ROWS_PER_SUB)],
                    c_out.at[pl.ds(c * NPAD + base, ROWS_PER_SUB)])


def _merge1_body(s_ref, c_ref, h_ref, p2_ref, p3_ref, hn_ref, cnt_ref):
    S = s_ref[0:NPAD, :] + s_ref[NPAD:2 * NPAD, :]
    cnt = c_ref[0:NPAD, :] + c_ref[NPAD:2 * NPAD, :]
    inv = 1.0 / jnp.maximum(cnt, 1.0)
    p2 = p2_ref[0, 0]
    p3 = p3_ref[0, 0]
    hn_ref[...] = jnp.tanh(p2 * S * inv + p3 * cnt * inv * h_ref[...])
    cnt_ref[...] = cnt


def _merge1_call(s2, c2, h, p2, p3):
    return pl.pallas_call(
        _merge1_body,
        out_shape=(jax.ShapeDtypeStruct((NPAD, B), jnp.float32),
                   jax.ShapeDtypeStruct((NPAD, B), jnp.float32)),
    )(s2, c2, h, p2, p3)


def _merge2_body(s_ref, cnt_ref, h_ref, p2_ref, p3_ref, hn_ref):
    S = s_ref[0:NPAD, :] + s_ref[NPAD:2 * NPAD, :]
    cnt = cnt_ref[...]
    inv = 1.0 / jnp.maximum(cnt, 1.0)
    p2 = p2_ref[0, 0]
    p3 = p3_ref[0, 0]
    hn_ref[...] = jnp.tanh(p2 * S * inv + p3 * cnt * inv * h_ref[...])


def _merge2_call(s2, cnt, h, p2, p3):
    return pl.pallas_call(
        _merge2_body,
        out_shape=jax.ShapeDtypeStruct((NPAD, B), jnp.float32),
    )(s2, cnt, h, p2, p3)


def _merge3_body(s_ref, cnt_ref, h_ref, p2_ref, p3_ref, w_ref, b_ref,
                 out_ref):
    S = s_ref[0:NPAD, :] + s_ref[NPAD:2 * NPAD, :]
    cnt = cnt_ref[...]
    inv = 1.0 / jnp.maximum(cnt, 1.0)
    p2 = p2_ref[0, 0]
    p3 = p3_ref[0, 0]
    h3 = jnp.tanh(p2 * S * inv + p3 * cnt * inv * h_ref[...])
    out = lax.dot_general(h3, w_ref[...], (((0,), (0,)), ((), ())),
                          preferred_element_type=jnp.float32)
    out_ref[...] = out + b_ref[0, 0]


def _merge3_call(s2, cnt, h, p2, p3, wT, b2):
    return pl.pallas_call(
        _merge3_body,
        out_shape=jax.ShapeDtypeStruct((B, 1), jnp.float32),
    )(s2, cnt, h, p2, p3, wT, b2)


def kernel(x, edge, edge_weight, param, p2s, p3s, W, b):
    pad = EPAD - E
    srcp = jnp.concatenate(
        [edge[0], jnp.full((pad,), N, jnp.int32)]).reshape(NW, NCH, CH)
    dstp = jnp.concatenate(
        [edge[1], jnp.full((pad,), N, jnp.int32)]).reshape(NW, NCH, CH)
    ones = jnp.ones((CH, B), jnp.float32)

    c1 = _sc_cnt(dstp, ones)
    h0 = _h0_call(x, param)
    h = jnp.concatenate(
        [h0, jnp.zeros((NPAD - N, B), jnp.float32)], axis=0)

    a2 = jnp.abs(p2s).reshape(NGCN, 1, 1).astype(jnp.float32)
    a3 = jnp.abs(p3s).reshape(NGCN, 1, 1).astype(jnp.float32)

    s1 = _sc_layer(h, srcp, dstp)
    h, cnt = _merge1_call(s1, c1, h, a2[0], a3[0])
    s2 = _sc_layer(h, srcp, dstp)
    h = _merge2_call(s2, cnt, h, a2[1], a3[1])
    s3 = _sc_layer(h, srcp, dstp)

    wT = jnp.concatenate(
        [jnp.transpose(W, (1, 0)),
         jnp.zeros((NPAD - N, 1), jnp.float32)], axis=0)
    b2 = b.reshape(1, 1).astype(jnp.float32)
    return _merge3_call(s3, cnt, h, a2[2], a3[2], wT, b2)


# speedup vs baseline: 14.9622x; 1.1692x over previous
import functools

import jax
import jax.numpy as jnp
from jax import lax
from jax.experimental import pallas as pl
from jax.experimental.pallas import tpu as pltpu
from jax.experimental.pallas import tpu_sc as plsc

N = 10000
E = 160000
B = 16
D = 256
NGCN = 3

NW = 32
CH = 512
NCH = 10
EPW = CH * NCH
EPAD = NW * EPW
NPAD = 10240
ROWS_PER_SUB = NPAD // 16

_NB = 400


def _h0_body(x_ref, p_ref, o_ref):
    xb = x_ref[...]
    p = jnp.abs(p_ref[...])
    acc = lax.dot_general(xb.reshape(B * _NB, D), p,
                          (((1,), (0,)), ((), ())),
                          preferred_element_type=jnp.float32)
    h = jnp.tanh(acc) * (1.0 / D)
    o_ref[...] = jnp.transpose(h.reshape(B, _NB), (1, 0))


def _h0_call(x, param):
    return pl.pallas_call(
        _h0_body,
        grid=(N // _NB,),
        in_specs=[
            pl.BlockSpec((B, _NB, D), lambda i: (0, i, 0)),
            pl.BlockSpec((D, 1), lambda i: (0, 0)),
        ],
        out_specs=pl.BlockSpec((_NB, B), lambda i: (i, 0)),
        out_shape=jax.ShapeDtypeStruct((N, B), jnp.float32),
    )(x, param)


_SC_MESH = plsc.VectorSubcoreMesh(core_axis_name="c", subcore_axis_name="s")
_SC_PARAMS = pltpu.CompilerParams(use_tc_tiling_on_sc=False)


def _start_gather(h_hbm, srcv, j, buf, sem):
    pltpu.async_copy(h_hbm.at[srcv.at[j]], buf, sem)


def _wait_gather(h_hbm, srcv, j, buf, sem):
    pltpu.make_async_copy(h_hbm.at[srcv.at[j]], buf, sem).wait()


def _zero_shared(zbuf, s_sh, base):
    for r in range(16):
        zbuf[r, :] = jnp.zeros((B,), jnp.float32)

    def zbody(k, carry):
        pltpu.sync_copy(zbuf, s_sh.at[pl.ds(base + k * 16, 16)])
        return carry

    lax.fori_loop(0, ROWS_PER_SUB // 16, zbody, 0)


@functools.partial(
    pl.kernel,
    mesh=_SC_MESH,
    compiler_params=_SC_PARAMS,
    out_type=jax.ShapeDtypeStruct((2 * NPAD, B), jnp.float32),
    scratch_types=[
        pltpu.VMEM((NCH, CH), jnp.int32),
        pltpu.VMEM((NCH, CH), jnp.int32),
        pltpu.VMEM((CH, B), jnp.float32),
        pltpu.VMEM((CH, B), jnp.float32),
        pltpu.VMEM((16, B), jnp.float32),
        pltpu.VMEM_SHARED((NPAD, B), jnp.float32),
        pltpu.SemaphoreType.DMA,
        pltpu.SemaphoreType.DMA,
    ],
)
def _sc_layer(h_hbm, src_hbm, dst_hbm, s_out, srcv, dstv, g0, g1, zbuf, s_sh,
              sem0, sem1):
    c = lax.axis_index("c")
    s = lax.axis_index("s")
    wid = s * 2 + c
    pltpu.sync_copy(src_hbm.at[wid], srcv)
    pltpu.sync_copy(dst_hbm.at[wid], dstv)
    base = s * ROWS_PER_SUB
    _zero_shared(zbuf, s_sh, base)
    plsc.subcore_barrier()

    _start_gather(h_hbm, srcv, 0, g0, sem0)
    ngrp = NCH // 2

    def group(g, carry):
        j0 = 2 * g
        _start_gather(h_hbm, srcv, j0 + 1, g1, sem1)
        _wait_gather(h_hbm, srcv, j0, g0, sem0)
        pltpu.sync_copy(g0, s_sh.at[dstv.at[j0]], add=True)

        @pl.when(g < ngrp - 1)
        def _():
            _start_gather(h_hbm, srcv, j0 + 2, g0, sem0)

        _wait_gather(h_hbm, srcv, j0 + 1, g1, sem1)
        pltpu.sync_copy(g1, s_sh.at[dstv.at[j0 + 1]], add=True)
        return carry

    lax.fori_loop(0, ngrp, group, 0)
    plsc.subcore_barrier()
    pltpu.sync_copy(s_sh.at[pl.ds(base, ROWS_PER_SUB)],
                    s_out.at[pl.ds(c * NPAD + base, ROWS_PER_SUB)])


@functools.partial(
    pl.kernel,
    mesh=_SC_MESH,
    compiler_params=_SC_PARAMS,
    out_type=jax.ShapeDtypeStruct((2 * NPAD,), jnp.float32),
    scratch_types=[
        pltpu.VMEM((NCH, CH), jnp.int32),
        pltpu.VMEM((CH,), jnp.float32),
        pltpu.VMEM((16,), jnp.float32),
        pltpu.VMEM_SHARED((NPAD,), jnp.float32),
        pltpu.SemaphoreType.DMA,
    ],
)
def _sc_cnt(dst_hbm, ones_hbm, c_out, dstv, obuf, zbuf, c_sh, sem):
    c = lax.axis_index("c")
    s = lax.axis_index("s")
    wid = s * 2 + c
    pltpu.sync_copy(dst_hbm.at[wid], dstv)
    pltpu.sync_copy(ones_hbm, obuf)
    zbuf[...] = jnp.zeros((16,), jnp.float32)
    base = s * ROWS_PER_SUB

    def zbody(k, carry):
        pltpu.sync_copy(zbuf, c_sh.at[pl.ds(base + k * 16, 16)])
        return carry

    lax.fori_loop(0, ROWS_PER_SUB // 16, zbody, 0)
    plsc.subcore_barrier()

    def ebody(j, carry):
        pltpu.sync_copy(obuf, c_sh.at[dstv.at[j]], add=True)
        return carry

    lax.fori_loop(0, NCH, ebody, 0)
    plsc.subcore_barrier()
    pltpu.sync_copy(c_sh.at[pl.ds(base, ROWS_PER_SUB)],
                    c_out.at[pl.ds(c * NPAD + base, ROWS_PER_SUB)])


_MROWS = NPAD // NW


@functools.partial(
    pl.kernel,
    mesh=_SC_MESH,
    compiler_params=_SC_PARAMS,
    out_type=jax.ShapeDtypeStruct((NPAD, B), jnp.float32),
    scratch_types=[
        pltpu.VMEM((_MROWS, B), jnp.float32),
        pltpu.VMEM((_MROWS, B), jnp.float32),
        pltpu.VMEM((_MROWS, B), jnp.float32),
        pltpu.VMEM((_MROWS,), jnp.float32),
        pltpu.VMEM((_MROWS,), jnp.float32),
        pltpu.VMEM((16,), jnp.float32),
        pltpu.VMEM((16,), jnp.float32),
    ],
)
def _sc_merge(s_hbm, c_hbm, h_hbm, p2_hbm, p3_hbm, hn_hbm,
              sv0, sv1, hv, cv0, cv1, p2v, p3v):
    c = lax.axis_index("c")
    s = lax.axis_index("s")
    wid = s * 2 + c
    base = wid * _MROWS
    pltpu.sync_copy(s_hbm.at[pl.ds(base, _MROWS)], sv0)
    pltpu.sync_copy(s_hbm.at[pl.ds(NPAD + base, _MROWS)], sv1)
    pltpu.sync_copy(h_hbm.at[pl.ds(base, _MROWS)], hv)
    pltpu.sync_copy(c_hbm.at[pl.ds(base, _MROWS)], cv0)
    pltpu.sync_copy(c_hbm.at[pl.ds(NPAD + base, _MROWS)], cv1)
    pltpu.sync_copy(p2_hbm, p2v)
    pltpu.sync_copy(p3_hbm, p3v)
    p2 = p2v[...]
    p3 = p3v[...]

    def gbody(g, carry):
        r0 = g * 16
        cw = cv0[pl.ds(r0, 16)] + cv1[pl.ds(r0, 16)]
        for k in range(16):
            r = r0 + k
            S = sv0[r, :] + sv1[r, :]
            cntv = jnp.full((16,), cw[k], jnp.float32)
            inv = 1.0 / jnp.maximum(cntv, 1.0)
            a = p2 * S * inv + p3 * cntv * inv * hv[r, :]
            e = jnp.exp(2.0 * a)
            hv[r, :] = 1.0 - 2.0 / (e + 1.0)
        return carry

    lax.fori_loop(0, _MROWS // 16, gbody, 0)
    pltpu.sync_copy(hv, hn_hbm.at[pl.ds(base, _MROWS)])


def _fin_body(h_ref, w_ref, b_ref, out_ref):
    out = lax.dot_general(h_ref[...], w_ref[...], (((0,), (0,)), ((), ())),
                          preferred_element_type=jnp.float32)
    out_ref[...] = out + b_ref[0, 0]


def _fin_call(h3, wT, b2):
    return pl.pallas_call(
        _fin_body,
        out_shape=jax.ShapeDtypeStruct((B, 1), jnp.float32),
    )(h3, wT, b2)


def kernel(x, edge, edge_weight, param, p2s, p3s, W, b):
    pad = EPAD - E
    srcp = jnp.concatenate(
        [edge[0], jnp.full((pad,), N, jnp.int32)]).reshape(NW, NCH, CH)
    dstp = jnp.concatenate(
        [edge[1], jnp.full((pad,), N, jnp.int32)]).reshape(NW, NCH, CH)
    ones = jnp.ones((CH,), jnp.float32)

    c1 = _sc_cnt(dstp, ones)
    h0 = _h0_call(x, param)
    h = jnp.concatenate(
        [h0, jnp.zeros((NPAD - N, B), jnp.float32)], axis=0)

    a2 = jnp.broadcast_to(jnp.abs(p2s).reshape(NGCN, 1),
                          (NGCN, 16)).astype(jnp.float32)
    a3 = jnp.broadcast_to(jnp.abs(p3s).reshape(NGCN, 1),
                          (NGCN, 16)).astype(jnp.float32)

    s1 = _sc_layer(h, srcp, dstp)
    h = _sc_merge(s1, c1, h, a2[0], a3[0])
    s2 = _sc_layer(h, srcp, dstp)
    h = _sc_merge(s2, c1, h, a2[1], a3[1])
    s3 = _sc_layer(h, srcp, dstp)
    h3 = _sc_merge(s3, c1, h, a2[2], a3[2])

    wT = jnp.concatenate(
        [jnp.transpose(W, (1, 0)),
         jnp.zeros((NPAD - N, 1), jnp.float32)], axis=0)
    b2 = b.reshape(1, 1).astype(jnp.float32)
    return _fin_call(h3, wT, b2)


# speedup vs baseline: 15.7139x; 1.0502x over previous
import functools

import jax
import jax.numpy as jnp
from jax import lax
from jax.experimental import pallas as pl
from jax.experimental.pallas import tpu as pltpu
from jax.experimental.pallas import tpu_sc as plsc

N = 10000
E = 160000
B = 16
D = 256
NGCN = 3

NW = 32
CH = 512
NCH = 10
EPW = CH * NCH
EPAD = NW * EPW
NPAD = 10240
ROWS_PER_SUB = NPAD // 16

_NB = 400


def _h0_body(x_ref, p_ref, o_ref):
    xb = x_ref[...]
    p = jnp.abs(p_ref[...])
    acc = lax.dot_general(xb.reshape(B * _NB, D), p,
                          (((1,), (0,)), ((), ())),
                          preferred_element_type=jnp.float32)
    h = jnp.tanh(acc) * (1.0 / D)
    o_ref[...] = jnp.transpose(h.reshape(B, _NB), (1, 0))


def _h0_call(x, param):
    return pl.pallas_call(
        _h0_body,
        grid=(N // _NB,),
        in_specs=[
            pl.BlockSpec((B, _NB, D), lambda i: (0, i, 0)),
            pl.BlockSpec((D, 1), lambda i: (0, 0)),
        ],
        out_specs=pl.BlockSpec((_NB, B), lambda i: (i, 0)),
        out_shape=jax.ShapeDtypeStruct((N, B), jnp.float32),
    )(x, param)


_SC_MESH = plsc.VectorSubcoreMesh(core_axis_name="c", subcore_axis_name="s")
_SC_PARAMS = pltpu.CompilerParams(use_tc_tiling_on_sc=False)


def _start_gather(h_hbm, srcv, j, buf, sem):
    pltpu.async_copy(h_hbm.at[srcv.at[j]], buf, sem)


def _wait_gather(h_hbm, srcv, j, buf, sem):
    pltpu.make_async_copy(h_hbm.at[srcv.at[j]], buf, sem).wait()


def _zero_shared(zbuf, s_sh, base):
    for r in range(16):
        zbuf[r, :] = jnp.zeros((B,), jnp.float32)

    def zbody(k, carry):
        pltpu.sync_copy(zbuf, s_sh.at[pl.ds(base + k * 16, 16)])
        return carry

    lax.fori_loop(0, ROWS_PER_SUB // 16, zbody, 0)


@functools.partial(
    pl.kernel,
    mesh=_SC_MESH,
    compiler_params=_SC_PARAMS,
    out_type=jax.ShapeDtypeStruct((2 * NPAD, B), jnp.float32),
    scratch_types=[
        pltpu.VMEM((NCH, CH), jnp.int32),
        pltpu.VMEM((NCH, CH), jnp.int32),
        pltpu.VMEM((CH, B), jnp.float32),
        pltpu.VMEM((CH, B), jnp.float32),
        pltpu.VMEM((16, B), jnp.float32),
        pltpu.VMEM_SHARED((NPAD, B), jnp.float32),
        pltpu.SemaphoreType.DMA,
        pltpu.SemaphoreType.DMA,
    ],
)
def _sc_layer(h_hbm, src_hbm, dst_hbm, s_out, srcv, dstv, g0, g1, zbuf, s_sh,
              sem0, sem1):
    c = lax.axis_index("c")
    s = lax.axis_index("s")
    wid = s * 2 + c
    pltpu.sync_copy(src_hbm.at[wid], srcv)
    pltpu.sync_copy(dst_hbm.at[wid], dstv)
    base = s * ROWS_PER_SUB
    _zero_shared(zbuf, s_sh, base)
    plsc.subcore_barrier()

    _start_gather(h_hbm, srcv, 0, g0, sem0)
    ngrp = NCH // 2

    def group(g, carry):
        j0 = 2 * g
        _start_gather(h_hbm, srcv, j0 + 1, g1, sem1)
        _wait_gather(h_hbm, srcv, j0, g0, sem0)
        pltpu.sync_copy(g0, s_sh.at[dstv.at[j0]], add=True)

        @pl.when(g < ngrp - 1)
        def _():
            _start_gather(h_hbm, srcv, j0 + 2, g0, sem0)

        _wait_gather(h_hbm, srcv, j0 + 1, g1, sem1)
        pltpu.sync_copy(g1, s_sh.at[dstv.at[j0 + 1]], add=True)
        return carry

    lax.fori_loop(0, ngrp, group, 0)
    plsc.subcore_barrier()
    pltpu.sync_copy(s_sh.at[pl.ds(base, ROWS_PER_SUB)],
                    s_out.at[pl.ds(c * NPAD + base, ROWS_PER_SUB)])


@functools.partial(
    pl.kernel,
    mesh=_SC_MESH,
    compiler_params=_SC_PARAMS,
    out_type=(jax.ShapeDtypeStruct((2 * NPAD, B), jnp.float32),
              jax.ShapeDtypeStruct((2 * NPAD,), jnp.float32)),
    scratch_types=[
        pltpu.VMEM((NCH, CH), jnp.int32),
        pltpu.VMEM((NCH, CH), jnp.int32),
        pltpu.VMEM((CH, B), jnp.float32),
        pltpu.VMEM((CH, B), jnp.float32),
        pltpu.VMEM((CH,), jnp.float32),
        pltpu.VMEM((16, B), jnp.float32),
        pltpu.VMEM((16,), jnp.float32),
        pltpu.VMEM_SHARED((NPAD, B), jnp.float32),
        pltpu.VMEM_SHARED((NPAD,), jnp.float32),
        pltpu.SemaphoreType.DMA,
        pltpu.SemaphoreType.DMA,
    ],
)
def _sc_layer1(h_hbm, src_hbm, dst_hbm, ones_hbm, s_out, c_out, srcv, dstv,
               g0, g1, obuf, zbuf, zbuf1, s_sh, c_sh, sem0, sem1):
    c = lax.axis_index("c")
    s = lax.axis_index("s")
    wid = s * 2 + c
    pltpu.sync_copy(src_hbm.at[wid], srcv)
    pltpu.sync_copy(dst_hbm.at[wid], dstv)
    pltpu.sync_copy(ones_hbm, obuf)
    base = s * ROWS_PER_SUB
    _zero_shared(zbuf, s_sh, base)
    zbuf1[...] = jnp.zeros((16,), jnp.float32)

    def zbody(k, carry):
        pltpu.sync_copy(zbuf1, c_sh.at[pl.ds(base + k * 16, 16)])
        return carry

    lax.fori_loop(0, ROWS_PER_SUB // 16, zbody, 0)
    plsc.subcore_barrier()

    _start_gather(h_hbm, srcv, 0, g0, sem0)
    ngrp = NCH // 2

    def group(g, carry):
        j0 = 2 * g
        _start_gather(h_hbm, srcv, j0 + 1, g1, sem1)
        _wait_gather(h_hbm, srcv, j0, g0, sem0)
        pltpu.sync_copy(g0, s_sh.at[dstv.at[j0]], add=True)
        pltpu.sync_copy(obuf, c_sh.at[dstv.at[j0]], add=True)

        @pl.when(g < ngrp - 1)
        def _():
            _start_gather(h_hbm, srcv, j0 + 2, g0, sem0)

        _wait_gather(h_hbm, srcv, j0 + 1, g1, sem1)
        pltpu.sync_copy(g1, s_sh.at[dstv.at[j0 + 1]], add=True)
        pltpu.sync_copy(obuf, c_sh.at[dstv.at[j0 + 1]], add=True)
        return carry

    lax.fori_loop(0, ngrp, group, 0)
    plsc.subcore_barrier()
    pltpu.sync_copy(s_sh.at[pl.ds(base, ROWS_PER_SUB)],
                    s_out.at[pl.ds(c * NPAD + base, ROWS_PER_SUB)])
    pltpu.sync_copy(c_sh.at[pl.ds(base, ROWS_PER_SUB)],
                    c_out.at[pl.ds(c * NPAD + base, ROWS_PER_SUB)])


_MROWS = NPAD // NW


@functools.partial(
    pl.kernel,
    mesh=_SC_MESH,
    compiler_params=_SC_PARAMS,
    out_type=jax.ShapeDtypeStruct((NPAD, B), jnp.float32),
    scratch_types=[
        pltpu.VMEM((_MROWS, B), jnp.float32),
        pltpu.VMEM((_MROWS, B), jnp.float32),
        pltpu.VMEM((_MROWS, B), jnp.float32),
        pltpu.VMEM((_MROWS,), jnp.float32),
        pltpu.VMEM((_MROWS,), jnp.float32),
        pltpu.VMEM((16,), jnp.float32),
        pltpu.VMEM((16,), jnp.float32),
    ],
)
def _sc_merge(s_hbm, c_hbm, h_hbm, p2_hbm, p3_hbm, hn_hbm,
              sv0, sv1, hv, cv0, cv1, p2v, p3v):
    c = lax.axis_index("c")
    s = lax.axis_index("s")
    wid = s * 2 + c
    base = wid * _MROWS
    pltpu.sync_copy(s_hbm.at[pl.ds(base, _MROWS)], sv0)
    pltpu.sync_copy(s_hbm.at[pl.ds(NPAD + base, _MROWS)], sv1)
    pltpu.sync_copy(h_hbm.at[pl.ds(base, _MROWS)], hv)
    pltpu.sync_copy(c_hbm.at[pl.ds(base, _MROWS)], cv0)
    pltpu.sync_copy(c_hbm.at[pl.ds(NPAD + base, _MROWS)], cv1)
    pltpu.sync_copy(p2_hbm, p2v)
    pltpu.sync_copy(p3_hbm, p3v)
    p2 = p2v[...]
    p3 = p3v[...]

    def gbody(g, carry):
        r0 = g * 16
        cw = cv0[pl.ds(r0, 16)] + cv1[pl.ds(r0, 16)]
        for k in range(16):
            r = r0 + k
            S = sv0[r, :] + sv1[r, :]
            cntv = jnp.full((16,), cw[k], jnp.float32)
            inv = 1.0 / jnp.maximum(cntv, 1.0)
            a = p2 * S * inv + p3 * cntv * inv * hv[r, :]
            e = jnp.exp(2.0 * a)
            hv[r, :] = 1.0 - 2.0 / (e + 1.0)
        return carry

    lax.fori_loop(0, _MROWS // 16, gbody, 0)
    pltpu.sync_copy(hv, hn_hbm.at[pl.ds(base, _MROWS)])


def _fin_body(h_ref, w_ref, b_ref, out_ref):
    out = lax.dot_general(h_ref[0:N, :], w_ref[...],
                          (((0,), (1,)), ((), ())),
                          preferred_element_type=jnp.float32)
    out_ref[...] = out + b_ref[0, 0]


def _fin_call(h3, wT, b2):
    return pl.pallas_call(
        _fin_body,
        out_shape=jax.ShapeDtypeStruct((B, 1), jnp.float32),
    )(h3, wT, b2)


def kernel(x, edge, edge_weight, param, p2s, p3s, W, b):
    pad = EPAD - E
    srcp = jnp.concatenate(
        [edge[0], jnp.full((pad,), N, jnp.int32)]).reshape(NW, NCH, CH)
    dstp = jnp.concatenate(
        [edge[1],
         N + (jnp.arange(pad, dtype=jnp.int32) % (NPAD - N))]
    ).reshape(NW, NCH, CH)
    ones = jnp.ones((CH,), jnp.float32)

    h0 = _h0_call(x, param)
    h = jnp.concatenate(
        [h0, jnp.zeros((NPAD - N, B), jnp.float32)], axis=0)

    a2 = jnp.broadcast_to(jnp.abs(p2s).reshape(NGCN, 1),
                          (NGCN, 16)).astype(jnp.float32)
    a3 = jnp.broadcast_to(jnp.abs(p3s).reshape(NGCN, 1),
                          (NGCN, 16)).astype(jnp.float32)

    s1, c1 = _sc_layer1(h, srcp, dstp, ones)
    h = _sc_merge(s1, c1, h, a2[0], a3[0])
    s2 = _sc_layer(h, srcp, dstp)
    h = _sc_merge(s2, c1, h, a2[1], a3[1])
    s3 = _sc_layer(h, srcp, dstp)
    h3 = _sc_merge(s3, c1, h, a2[2], a3[2])

    b2 = b.reshape(1, 1).astype(jnp.float32)
    return _fin_call(h3, W, b2)


# speedup vs baseline: 15.7256x; 1.0007x over previous
import functools

import jax
import jax.numpy as jnp
from jax import lax
from jax.experimental import pallas as pl
from jax.experimental.pallas import tpu as pltpu
from jax.experimental.pallas import tpu_sc as plsc

N = 10000
E = 160000
B = 16
D = 256
NGCN = 3

NW = 32
CH = 512
NCH = 10
EPW = CH * NCH
EPAD = NW * EPW
NPAD = 10240
ROWS_PER_SUB = NPAD // 16

_NB = 400


def _h0_body(x_ref, p_ref, o_ref):
    xb = x_ref[...]
    p = jnp.abs(p_ref[...])
    acc = lax.dot_general(xb.reshape(B * _NB, D), p,
                          (((1,), (0,)), ((), ())),
                          preferred_element_type=jnp.float32)
    h = jnp.tanh(acc) * (1.0 / D)
    o_ref[...] = jnp.transpose(h.reshape(B, _NB), (1, 0))


def _h0_call(x, param):
    return pl.pallas_call(
        _h0_body,
        grid=(N // _NB,),
        in_specs=[
            pl.BlockSpec((B, _NB, D), lambda i: (0, i, 0)),
            pl.BlockSpec((D, 1), lambda i: (0, 0)),
        ],
        out_specs=pl.BlockSpec((_NB, B), lambda i: (i, 0)),
        out_shape=jax.ShapeDtypeStruct((N, B), jnp.float32),
    )(x, param)


_SC_MESH = plsc.VectorSubcoreMesh(core_axis_name="c", subcore_axis_name="s")
_SC_PARAMS = pltpu.CompilerParams(use_tc_tiling_on_sc=False)

_GSEMS = [pltpu.SemaphoreType.DMA] * NCH


@functools.partial(
    pl.kernel,
    mesh=_SC_MESH,
    compiler_params=_SC_PARAMS,
    out_type=jax.ShapeDtypeStruct((2 * NPAD, B), jnp.float32),
    scratch_types=[
        pltpu.VMEM((NCH, CH), jnp.int32),
        pltpu.VMEM((NCH, CH), jnp.int32),
        pltpu.VMEM((NCH, CH, B), jnp.float32),
        pltpu.VMEM((ROWS_PER_SUB, B), jnp.float32),
        pltpu.VMEM_SHARED((NPAD, B), jnp.float32),
        pltpu.SemaphoreType.DMA,
    ] + _GSEMS,
)
def _sc_layer(h_hbm, src_hbm, dst_hbm, s_out, srcv, dstv, gb, zbuf, s_sh,
              ssem, *gsems):
    c = lax.axis_index("c")
    s = lax.axis_index("s")
    wid = s * 2 + c
    pltpu.sync_copy(src_hbm.at[wid], srcv)
    pltpu.sync_copy(dst_hbm.at[wid], dstv)
    base = s * ROWS_PER_SUB

    for j in range(NCH):
        pltpu.async_copy(h_hbm.at[srcv.at[j]], gb.at[j], gsems[j])

    zbuf[...] = jnp.zeros((ROWS_PER_SUB, B), jnp.float32)
    pltpu.sync_copy(zbuf, s_sh.at[pl.ds(base, ROWS_PER_SUB)])
    plsc.subcore_barrier()

    for j in range(NCH):
        pltpu.make_async_copy(h_hbm.at[srcv.at[j]], gb.at[j], gsems[j]).wait()
        pltpu.async_copy(gb.at[j], s_sh.at[dstv.at[j]], ssem, add=True)

    for j in range(NCH):
        pltpu.make_async_copy(gb.at[j], s_sh.at[dstv.at[j]], ssem).wait()
    plsc.subcore_barrier()
    pltpu.sync_copy(s_sh.at[pl.ds(base, ROWS_PER_SUB)],
                    s_out.at[pl.ds(c * NPAD + base, ROWS_PER_SUB)])


@functools.partial(
    pl.kernel,
    mesh=_SC_MESH,
    compiler_params=_SC_PARAMS,
    out_type=jax.ShapeDtypeStruct((2 * NPAD,), jnp.float32),
    scratch_types=[
        pltpu.VMEM((NCH, CH), jnp.int32),
        pltpu.VMEM((CH,), jnp.float32),
        pltpu.VMEM((ROWS_PER_SUB,), jnp.float32),
        pltpu.VMEM_SHARED((NPAD,), jnp.float32),
        pltpu.SemaphoreType.DMA,
    ],
)
def _sc_cnt(dst_hbm, ones_hbm, c_out, dstv, obuf, zbuf, c_sh, ssem):
    c = lax.axis_index("c")
    s = lax.axis_index("s")
    wid = s * 2 + c
    pltpu.sync_copy(dst_hbm.at[wid], dstv)
    pltpu.sync_copy(ones_hbm, obuf)
    base = s * ROWS_PER_SUB
    zbuf[...] = jnp.zeros((ROWS_PER_SUB,), jnp.float32)
    pltpu.sync_copy(zbuf, c_sh.at[pl.ds(base, ROWS_PER_SUB)])
    plsc.subcore_barrier()

    for j in range(NCH):
        pltpu.async_copy(obuf, c_sh.at[dstv.at[j]], ssem, add=True)
    for j in range(NCH):
        pltpu.make_async_copy(obuf, c_sh.at[dstv.at[j]], ssem).wait()
    plsc.subcore_barrier()
    pltpu.sync_copy(c_sh.at[pl.ds(base, ROWS_PER_SUB)],
                    c_out.at[pl.ds(c * NPAD + base, ROWS_PER_SUB)])


_MROWS = NPAD // NW


@functools.partial(
    pl.kernel,
    mesh=_SC_MESH,
    compiler_params=_SC_PARAMS,
    out_type=jax.ShapeDtypeStruct((NPAD, B), jnp.float32),
    scratch_types=[
        pltpu.VMEM((_MROWS, B), jnp.float32),
        pltpu.VMEM((_MROWS, B), jnp.float32),
        pltpu.VMEM((_MROWS, B), jnp.float32),
        pltpu.VMEM((_MROWS,), jnp.float32),
        pltpu.VMEM((_MROWS,), jnp.float32),
        pltpu.VMEM((16,), jnp.float32),
        pltpu.VMEM((16,), jnp.float32),
    ],
)
def _sc_merge(s_hbm, c_hbm, h_hbm, p2_hbm, p3_hbm, hn_hbm,
              sv0, sv1, hv, cv0, cv1, p2v, p3v):
    c = lax.axis_index("c")
    s = lax.axis_index("s")
    wid = s * 2 + c
    base = wid * _MROWS
    pltpu.sync_copy(s_hbm.at[pl.ds(base, _MROWS)], sv0)
    pltpu.sync_copy(s_hbm.at[pl.ds(NPAD + base, _MROWS)], sv1)
    pltpu.sync_copy(h_hbm.at[pl.ds(base, _MROWS)], hv)
    pltpu.sync_copy(c_hbm.at[pl.ds(base, _MROWS)], cv0)
    pltpu.sync_copy(c_hbm.at[pl.ds(NPAD + base, _MROWS)], cv1)
    pltpu.sync_copy(p2_hbm, p2v)
    pltpu.sync_copy(p3_hbm, p3v)
    p2 = p2v[...]
    p3 = p3v[...]

    def gbody(g, carry):
        r0 = g * 16
        cw = cv0[pl.ds(r0, 16)] + cv1[pl.ds(r0, 16)]
        for k in range(16):
            r = r0 + k
            S = sv0[r, :] + sv1[r, :]
            cntv = jnp.full((16,), cw[k], jnp.float32)
            inv = 1.0 / jnp.maximum(cntv, 1.0)
            a = p2 * S * inv + p3 * cntv * inv * hv[r, :]
            e = jnp.exp(2.0 * a)
            hv[r, :] = 1.0 - 2.0 / (e + 1.0)
        return carry

    lax.fori_loop(0, _MROWS // 16, gbody, 0)
    pltpu.sync_copy(hv, hn_hbm.at[pl.ds(base, _MROWS)])


def _fin_body(h_ref, w_ref, b_ref, out_ref):
    out = lax.dot_general(h_ref[0:N, :], w_ref[...],
                          (((0,), (1,)), ((), ())),
                          preferred_element_type=jnp.float32)
    out_ref[...] = out + b_ref[0, 0]


def _fin_call(h3, wT, b2):
    return pl.pallas_call(
        _fin_body,
        out_shape=jax.ShapeDtypeStruct((B, 1), jnp.float32),
    )(h3, wT, b2)


def kernel(x, edge, edge_weight, param, p2s, p3s, W, b):
    pad = EPAD - E
    srcp = jnp.concatenate(
        [edge[0], jnp.full((pad,), N, jnp.int32)]).reshape(NW, NCH, CH)
    dstp = jnp.concatenate(
        [edge[1],
         N + (jnp.arange(pad, dtype=jnp.int32) % (NPAD - N))]
    ).reshape(NW, NCH, CH)
    ones = jnp.ones((CH,), jnp.float32)

    c1 = _sc_cnt(dstp, ones)
    h0 = _h0_call(x, param)
    h = jnp.concatenate(
        [h0, jnp.zeros((NPAD - N, B), jnp.float32)], axis=0)

    a2 = jnp.broadcast_to(jnp.abs(p2s).reshape(NGCN, 1),
                          (NGCN, 16)).astype(jnp.float32)
    a3 = jnp.broadcast_to(jnp.abs(p3s).reshape(NGCN, 1),
                          (NGCN, 16)).astype(jnp.float32)

    s1 = _sc_layer(h, srcp, dstp)
    h = _sc_merge(s1, c1, h, a2[0], a3[0])
    s2 = _sc_layer(h, srcp, dstp)
    h = _sc_merge(s2, c1, h, a2[1], a3[1])
    s3 = _sc_layer(h, srcp, dstp)
    h3 = _sc_merge(s3, c1, h, a2[2], a3[2])

    b2 = b.reshape(1, 1).astype(jnp.float32)
    return _fin_call(h3, W, b2)


# speedup vs baseline: 15.9441x; 1.0139x over previous
import functools

import jax
import jax.numpy as jnp
from jax import lax
from jax.experimental import pallas as pl
from jax.experimental.pallas import tpu as pltpu
from jax.experimental.pallas import tpu_sc as plsc

N = 10000
E = 160000
B = 16
D = 256
NGCN = 3

NW = 32
CH = 512
NCH = 10
EPW = CH * NCH
EPAD = NW * EPW
NPAD = 10240
ROWS_PER_SUB = NPAD // 16

_EXTRA = EPAD - E


def _prep_body(e_ref, src_ref, dst_ref):
    src = e_ref[0:1, :]
    dst = e_ref[1:2, :]
    psrc = jnp.full((1, _EXTRA), N, jnp.int32)
    pdst = N + lax.rem(lax.broadcasted_iota(jnp.int32, (1, _EXTRA), 1),
                       NPAD - N)
    src_ref[...] = jnp.concatenate([src, psrc], axis=1).reshape(NW, NCH, CH)
    dst_ref[...] = jnp.concatenate([dst, pdst], axis=1).reshape(NW, NCH, CH)


def _prep_call(edge):
    return pl.pallas_call(
        _prep_body,
        out_shape=(jax.ShapeDtypeStruct((NW, NCH, CH), jnp.int32),
                   jax.ShapeDtypeStruct((NW, NCH, CH), jnp.int32)),
    )(edge)


_NB = 1000


def _h0_body(x_ref, p_ref, o_ref):
    xb = x_ref[...]
    p = jnp.abs(p_ref[...])
    acc = lax.dot_general(xb.reshape(B * _NB, D), p,
                          (((1,), (0,)), ((), ())),
                          preferred_element_type=jnp.float32)
    h = jnp.tanh(acc) * (1.0 / D)
    o_ref[...] = jnp.transpose(h.reshape(B, _NB), (1, 0))


def _h0_call(x, param):
    return pl.pallas_call(
        _h0_body,
        grid=(N // _NB,),
        in_specs=[
            pl.BlockSpec((B, _NB, D), lambda i: (0, i, 0)),
            pl.BlockSpec((D, 1), lambda i: (0, 0)),
        ],
        out_specs=pl.BlockSpec((_NB, B), lambda i: (i, 0)),
        out_shape=jax.ShapeDtypeStruct((N, B), jnp.float32),
    )(x, param)


_SC_MESH = plsc.VectorSubcoreMesh(core_axis_name="c", subcore_axis_name="s")
_SC_PARAMS = pltpu.CompilerParams(use_tc_tiling_on_sc=False)

_GSEMS = [pltpu.SemaphoreType.DMA] * NCH


@functools.partial(
    pl.kernel,
    mesh=_SC_MESH,
    compiler_params=_SC_PARAMS,
    out_type=jax.ShapeDtypeStruct((2 * NPAD, B), jnp.float32),
    scratch_types=[
        pltpu.VMEM((NCH, CH), jnp.int32),
        pltpu.VMEM((NCH, CH), jnp.int32),
        pltpu.VMEM((NCH, CH, B), jnp.float32),
        pltpu.VMEM((ROWS_PER_SUB, B), jnp.float32),
        pltpu.VMEM_SHARED((NPAD, B), jnp.float32),
        pltpu.SemaphoreType.DMA,
    ] + _GSEMS,
)
def _sc_layer(h_hbm, src_hbm, dst_hbm, s_out, srcv, dstv, gb, zbuf, s_sh,
              ssem, *gsems):
    c = lax.axis_index("c")
    s = lax.axis_index("s")
    wid = s * 2 + c
    pltpu.sync_copy(src_hbm.at[wid], srcv)
    pltpu.sync_copy(dst_hbm.at[wid], dstv)
    base = s * ROWS_PER_SUB

    for j in range(NCH):
        pltpu.async_copy(h_hbm.at[srcv.at[j]], gb.at[j], gsems[j])

    zbuf[...] = jnp.zeros((ROWS_PER_SUB, B), jnp.float32)
    pltpu.sync_copy(zbuf, s_sh.at[pl.ds(base, ROWS_PER_SUB)])
    plsc.subcore_barrier()

    for j in range(NCH):
        pltpu.make_async_copy(h_hbm.at[srcv.at[j]], gb.at[j], gsems[j]).wait()
        pltpu.async_copy(gb.at[j], s_sh.at[dstv.at[j]], ssem, add=True)

    for j in range(NCH):
        pltpu.make_async_copy(gb.at[j], s_sh.at[dstv.at[j]], ssem).wait()
    plsc.subcore_barrier()
    pltpu.sync_copy(s_sh.at[pl.ds(base, ROWS_PER_SUB)],
                    s_out.at[pl.ds(c * NPAD + base, ROWS_PER_SUB)])


@functools.partial(
    pl.kernel,
    mesh=_SC_MESH,
    compiler_params=_SC_PARAMS,
    out_type=jax.ShapeDtypeStruct((2 * NPAD,), jnp.float32),
    scratch_types=[
        pltpu.VMEM((NCH, CH), jnp.int32),
        pltpu.VMEM((CH,), jnp.float32),
        pltpu.VMEM((ROWS_PER_SUB,), jnp.float32),
        pltpu.VMEM_SHARED((NPAD,), jnp.float32),
        pltpu.SemaphoreType.DMA,
    ],
)
def _sc_cnt(dst_hbm, c_out, dstv, obuf, zbuf, c_sh, ssem):
    c = lax.axis_index("c")
    s = lax.axis_index("s")
    wid = s * 2 + c
    pltpu.sync_copy(dst_hbm.at[wid], dstv)
    obuf[...] = jnp.full((CH,), 1.0, jnp.float32)
    base = s * ROWS_PER_SUB
    zbuf[...] = jnp.zeros((ROWS_PER_SUB,), jnp.float32)
    pltpu.sync_copy(zbuf, c_sh.at[pl.ds(base, ROWS_PER_SUB)])
    plsc.subcore_barrier()

    for j in range(NCH):
        pltpu.async_copy(obuf, c_sh.at[dstv.at[j]], ssem, add=True)
    for j in range(NCH):
        pltpu.make_async_copy(obuf, c_sh.at[dstv.at[j]], ssem).wait()
    plsc.subcore_barrier()
    pltpu.sync_copy(c_sh.at[pl.ds(base, ROWS_PER_SUB)],
                    c_out.at[pl.ds(c * NPAD + base, ROWS_PER_SUB)])


_MROWS = NPAD // NW


def _make_merge(li):
    @functools.partial(
        pl.kernel,
        mesh=_SC_MESH,
        compiler_params=_SC_PARAMS,
        out_type=jax.ShapeDtypeStruct((NPAD, B), jnp.float32),
        scratch_types=[
            pltpu.VMEM((_MROWS, B), jnp.float32),
            pltpu.VMEM((_MROWS, B), jnp.float32),
            pltpu.VMEM((_MROWS, B), jnp.float32),
            pltpu.VMEM((_MROWS,), jnp.float32),
            pltpu.VMEM((_MROWS,), jnp.float32),
            pltpu.VMEM((16,), jnp.float32),
            pltpu.VMEM((16,), jnp.float32),
        ],
    )
    def _merge(s_hbm, c_hbm, h_hbm, p2_hbm, p3_hbm, hn_hbm,
               sv0, sv1, hv, cv0, cv1, p2v, p3v):
        c = lax.axis_index("c")
        s = lax.axis_index("s")
        wid = s * 2 + c
        base = wid * _MROWS
        pltpu.sync_copy(s_hbm.at[pl.ds(base, _MROWS)], sv0)
        pltpu.sync_copy(s_hbm.at[pl.ds(NPAD + base, _MROWS)], sv1)
        pltpu.sync_copy(h_hbm.at[pl.ds(base, _MROWS)], hv)
        pltpu.sync_copy(c_hbm.at[pl.ds(base, _MROWS)], cv0)
        pltpu.sync_copy(c_hbm.at[pl.ds(NPAD + base, _MROWS)], cv1)
        pltpu.sync_copy(p2_hbm, p2v)
        pltpu.sync_copy(p3_hbm, p3v)
        p2a = p2v[...]
        p3a = p3v[...]
        p2 = jnp.full((16,), p2a[li], jnp.float32)
        p3 = jnp.full((16,), p3a[li], jnp.float32)

        def gbody(g, carry):
            r0 = g * 16
            cw = cv0[pl.ds(r0, 16)] + cv1[pl.ds(r0, 16)]
            for k in range(16):
                r = r0 + k
                S = sv0[r, :] + sv1[r, :]
                cntv = jnp.full((16,), cw[k], jnp.float32)
                inv = 1.0 / jnp.maximum(cntv, 1.0)
                a = p2 * S * inv + p3 * cntv * inv * hv[r, :]
                e = jnp.exp(2.0 * a)
                hv[r, :] = 1.0 - 2.0 / (e + 1.0)
            return carry

        lax.fori_loop(0, _MROWS // 16, gbody, 0)
        pltpu.sync_copy(hv, hn_hbm.at[pl.ds(base, _MROWS)])

    return _merge


_sc_merge0 = _make_merge(0)
_sc_merge1 = _make_merge(1)


@functools.partial(
    pl.kernel,
    mesh=_SC_MESH,
    compiler_params=_SC_PARAMS,
    out_type=jax.ShapeDtypeStruct((NW, B), jnp.float32),
    scratch_types=[
        pltpu.VMEM((_MROWS, B), jnp.float32),
        pltpu.VMEM((_MROWS, B), jnp.float32),
        pltpu.VMEM((_MROWS, B), jnp.float32),
        pltpu.VMEM((_MROWS,), jnp.float32),
        pltpu.VMEM((_MROWS,), jnp.float32),
        pltpu.VMEM((_MROWS,), jnp.float32),
        pltpu.VMEM((16,), jnp.float32),
        pltpu.VMEM((16,), jnp.float32),
        pltpu.VMEM((16,), jnp.float32),
    ],
)
def _sc_merge_fin(s_hbm, c_hbm, h_hbm, p2_hbm, p3_hbm, w_hbm, d_out,
                  sv0, sv1, hv, cv0, cv1, wv, p2v, p3v, accv):
    c = lax.axis_index("c")
    s = lax.axis_index("s")
    wid = s * 2 + c
    base = wid * _MROWS
    pltpu.sync_copy(s_hbm.at[pl.ds(base, _MROWS)], sv0)
    pltpu.sync_copy(s_hbm.at[pl.ds(NPAD + base, _MROWS)], sv1)
    pltpu.sync_copy(h_hbm.at[pl.ds(base, _MROWS)], hv)
    pltpu.sync_copy(c_hbm.at[pl.ds(base, _MROWS)], cv0)
    pltpu.sync_copy(c_hbm.at[pl.ds(NPAD + base, _MROWS)], cv1)
    pltpu.sync_copy(w_hbm.at[pl.ds(base, _MROWS)], wv)
    pltpu.sync_copy(p2_hbm, p2v)
    pltpu.sync_copy(p3_hbm, p3v)
    p2a = p2v[...]
    p3a = p3v[...]
    p2 = jnp.full((16,), p2a[2], jnp.float32)
    p3 = jnp.full((16,), p3a[2], jnp.float32)

    def gbody(g, acc):
        r0 = g * 16
        cw = cv0[pl.ds(r0, 16)] + cv1[pl.ds(r0, 16)]
        ww = wv[pl.ds(r0, 16)]
        for k in range(16):
            r = r0 + k
            S = sv0[r, :] + sv1[r, :]
            cntv = jnp.full((16,), cw[k], jnp.float32)
            inv = 1.0 / jnp.maximum(cntv, 1.0)
            a = p2 * S * inv + p3 * cntv * inv * hv[r, :]
            e = jnp.exp(2.0 * a)
            h3 = 1.0 - 2.0 / (e + 1.0)
            acc = acc + h3 * jnp.full((16,), ww[k], jnp.float32)
        return acc

    acc = lax.fori_loop(0, _MROWS // 16, gbody,
                        jnp.zeros((16,), jnp.float32))
    accv[...] = acc
    pltpu.sync_copy(accv, d_out.at[wid])


def _fin_body(d_ref, b_ref, out_ref):
    out_ref[...] = jnp.sum(d_ref[...], axis=0).reshape(B, 1) + b_ref[0, 0]


def _fin_call(dots, b2):
    return pl.pallas_call(
        _fin_body,
        out_shape=jax.ShapeDtypeStruct((B, 1), jnp.float32),
    )(dots, b2)


def kernel(x, edge, edge_weight, param, p2s, p3s, W, b):
    srcp, dstp = _prep_call(edge)
    c1 = _sc_cnt(dstp)
    h0 = _h0_call(x, param)
    h = jnp.concatenate(
        [h0, jnp.zeros((NPAD - N, B), jnp.float32)], axis=0)

    zpad = jnp.zeros((16 - NGCN,), jnp.float32)
    ab2 = jnp.concatenate([jnp.abs(p2s).reshape(NGCN), zpad])
    ab3 = jnp.concatenate([jnp.abs(p3s).reshape(NGCN), zpad])
    wpad = jnp.concatenate([W[0], jnp.zeros((NPAD - N,), jnp.float32)])

    s1 = _sc_layer(h, srcp, dstp)
    h = _sc_merge0(s1, c1, h, ab2, ab3)
    s2 = _sc_layer(h, srcp, dstp)
    h = _sc_merge1(s2, c1, h, ab2, ab3)
    s3 = _sc_layer(h, srcp, dstp)
    dots = _sc_merge_fin(s3, c1, h, ab2, ab3, wpad)

    b2 = b.reshape(1, 1).astype(jnp.float32)
    return _fin_call(dots, b2)


# speedup vs baseline: 16.8348x; 1.0559x over previous
import functools

import jax
import jax.numpy as jnp
from jax import lax
from jax.experimental import pallas as pl
from jax.experimental.pallas import tpu as pltpu
from jax.experimental.pallas import tpu_sc as plsc

N = 10000
E = 160000
B = 16
D = 256
NGCN = 3

CH = 512
NCH0 = 14
NCH1 = 6
NPAIR = NCH0 + NCH1
NW = 32
EPAD = 16 * NPAIR * CH
NPAD = 10240
ROWS_PER_SUB = NPAD // 16
NBUF = 8

_EXTRA = EPAD - E


def _split_edges(row, fill):
    a = row.reshape(16, NPAIR * CH)
    ev = a[:, :NCH0 * CH].reshape(16, 1, NCH0, CH)
    od = jnp.concatenate(
        [a[:, NCH0 * CH:].reshape(16, NCH1, CH),
         fill.reshape(16, NCH0 - NCH1, CH)], axis=1).reshape(16, 1, NCH0, CH)
    return jnp.concatenate([ev, od], axis=1).reshape(NW, NCH0, CH)


def _prep_body(e_ref, src_ref, dst_ref):
    psrc = jnp.full((1, _EXTRA), N, jnp.int32)
    pdst = N + lax.rem(lax.broadcasted_iota(jnp.int32, (1, _EXTRA), 1),
                       NPAD - N)
    srcfull = jnp.concatenate([e_ref[0:1, :], psrc], axis=1)
    dstfull = jnp.concatenate([e_ref[1:2, :], pdst], axis=1)
    fill = jnp.full((16 * (NCH0 - NCH1) * CH,), N, jnp.int32)
    src_ref[...] = _split_edges(srcfull, fill)
    dst_ref[...] = _split_edges(dstfull, fill)


def _prep_call(edge):
    return pl.pallas_call(
        _prep_body,
        out_shape=(jax.ShapeDtypeStruct((NW, NCH0, CH), jnp.int32),
                   jax.ShapeDtypeStruct((NW, NCH0, CH), jnp.int32)),
    )(edge)


_NB = 1000


def _h0_body(x_ref, p_ref, o_ref):
    xb = x_ref[...]
    p = jnp.abs(p_ref[...])
    acc = lax.dot_general(xb.reshape(B * _NB, D), p,
                          (((1,), (0,)), ((), ())),
                          preferred_element_type=jnp.float32)
    h = jnp.tanh(acc) * (1.0 / D)
    o_ref[...] = jnp.transpose(h.reshape(B, _NB), (1, 0))


def _h0_call(x, param):
    return pl.pallas_call(
        _h0_body,
        grid=(N // _NB,),
        in_specs=[
            pl.BlockSpec((B, _NB, D), lambda i: (0, i, 0)),
            pl.BlockSpec((D, 1), lambda i: (0, 0)),
        ],
        out_specs=pl.BlockSpec((_NB, B), lambda i: (i, 0)),
        out_shape=jax.ShapeDtypeStruct((N, B), jnp.float32),
    )(x, param)


_SC_MESH = plsc.VectorSubcoreMesh(core_axis_name="c", subcore_axis_name="s")
_SC_PARAMS = pltpu.CompilerParams(use_tc_tiling_on_sc=False)

_GSEMS = [pltpu.SemaphoreType.DMA] * NBUF
_SSEMS = [pltpu.SemaphoreType.DMA] * NBUF


@functools.partial(
    pl.kernel,
    mesh=_SC_MESH,
    compiler_params=_SC_PARAMS,
    out_type=jax.ShapeDtypeStruct((2 * NPAD, B), jnp.float32),
    scratch_types=[
        pltpu.VMEM((NCH0, CH), jnp.int32),
        pltpu.VMEM((NCH0, CH), jnp.int32),
        pltpu.VMEM((NBUF, CH, B), jnp.float32),
        pltpu.VMEM((ROWS_PER_SUB, B), jnp.float32),
        pltpu.VMEM_SHARED((NPAD, B), jnp.float32),
    ] + _GSEMS + _SSEMS,
)
def _sc_layer(h_hbm, src_hbm, dst_hbm, s_out, srcv, dstv, gb, zbuf, s_sh,
              *sems):
    gsems = sems[:NBUF]
    ssems = sems[NBUF:]
    c = lax.axis_index("c")
    s = lax.axis_index("s")
    wid = s * 2 + c
    nch = NCH0 - (NCH0 - NCH1) * c
    pltpu.sync_copy(src_hbm.at[wid], srcv)
    pltpu.sync_copy(dst_hbm.at[wid], dstv)
    base = s * ROWS_PER_SUB

    for j in range(NBUF):
        @pl.when(j < nch)
        def _():
            pltpu.async_copy(h_hbm.at[srcv.at[j]], gb.at[j], gsems[j])

    zbuf[...] = jnp.zeros((ROWS_PER_SUB, B), jnp.float32)
    pltpu.sync_copy(zbuf, s_sh.at[pl.ds(base, ROWS_PER_SUB)])
    plsc.subcore_barrier()

    for j in range(NCH0):
        b = j % NBUF

        @pl.when(j < nch)
        def _():
            pltpu.make_async_copy(h_hbm.at[srcv.at[j]], gb.at[b],
                                  gsems[b]).wait()
            pltpu.async_copy(gb.at[b], s_sh.at[dstv.at[j]], ssems[b],
                             add=True)

        nj = j + NBUF
        if nj < NCH0:
            nb = nj % NBUF

            @pl.when(nj < nch)
            def _():
                pltpu.make_async_copy(gb.at[nb], s_sh.at[dstv.at[j]],
                                      ssems[nb]).wait()
                pltpu.async_copy(h_hbm.at[srcv.at[nj]], gb.at[nb], gsems[nb])

    for j in range(NCH0):
        b = j % NBUF

        @pl.when((j < nch) & (j >= nch - NBUF))
        def _():
            pltpu.make_async_copy(gb.at[b], s_sh.at[dstv.at[j]],
                                  ssems[b]).wait()
    plsc.subcore_barrier()
    pltpu.sync_copy(s_sh.at[pl.ds(base, ROWS_PER_SUB)],
                    s_out.at[pl.ds(c * NPAD + base, ROWS_PER_SUB)])


@functools.partial(
    pl.kernel,
    mesh=_SC_MESH,
    compiler_params=_SC_PARAMS,
    out_type=jax.ShapeDtypeStruct((2 * NPAD,), jnp.float32),
    scratch_types=[
        pltpu.VMEM((NCH0, CH), jnp.int32),
        pltpu.VMEM((CH,), jnp.float32),
        pltpu.VMEM((ROWS_PER_SUB,), jnp.float32),
        pltpu.VMEM_SHARED((NPAD,), jnp.float32),
        pltpu.SemaphoreType.DMA,
    ],
)
def _sc_cnt(dst_hbm, c_out, dstv, obuf, zbuf, c_sh, ssem):
    c = lax.axis_index("c")
    s = lax.axis_index("s")
    wid = s * 2 + c
    nch = NCH0 - (NCH0 - NCH1) * c
    pltpu.sync_copy(dst_hbm.at[wid], dstv)
    obuf[...] = jnp.full((CH,), 1.0, jnp.float32)
    base = s * ROWS_PER_SUB
    zbuf[...] = jnp.zeros((ROWS_PER_SUB,), jnp.float32)
    pltpu.sync_copy(zbuf, c_sh.at[pl.ds(base, ROWS_PER_SUB)])
    plsc.subcore_barrier()

    for j in range(NCH0):
        @pl.when(j < nch)
        def _():
            pltpu.async_copy(obuf, c_sh.at[dstv.at[j]], ssem, add=True)
    for j in range(NCH0):
        @pl.when(j < nch)
        def _():
            pltpu.make_async_copy(obuf, c_sh.at[dstv.at[j]], ssem).wait()
    plsc.subcore_barrier()
    pltpu.sync_copy(c_sh.at[pl.ds(base, ROWS_PER_SUB)],
                    c_out.at[pl.ds(c * NPAD + base, ROWS_PER_SUB)])


_MROWS = NPAD // NW


def _make_merge(li):
    @functools.partial(
        pl.kernel,
        mesh=_SC_MESH,
        compiler_params=_SC_PARAMS,
        out_type=jax.ShapeDtypeStruct((NPAD, B), jnp.float32),
        scratch_types=[
            pltpu.VMEM((_MROWS, B), jnp.float32),
            pltpu.VMEM((_MROWS, B), jnp.float32),
            pltpu.VMEM((_MROWS, B), jnp.float32),
            pltpu.VMEM((_MROWS,), jnp.float32),
            pltpu.VMEM((_MROWS,), jnp.float32),
            pltpu.VMEM((16,), jnp.float32),
            pltpu.VMEM((16,), jnp.float32),
        ],
    )
    def _merge(s_hbm, c_hbm, h_hbm, p2_hbm, p3_hbm, hn_hbm,
               sv0, sv1, hv, cv0, cv1, p2v, p3v):
        c = lax.axis_index("c")
        s = lax.axis_index("s")
        wid = s * 2 + c
        base = wid * _MROWS
        pltpu.sync_copy(s_hbm.at[pl.ds(base, _MROWS)], sv0)
        pltpu.sync_copy(s_hbm.at[pl.ds(NPAD + base, _MROWS)], sv1)
        pltpu.sync_copy(h_hbm.at[pl.ds(base, _MROWS)], hv)
        pltpu.sync_copy(c_hbm.at[pl.ds(base, _MROWS)], cv0)
        pltpu.sync_copy(c_hbm.at[pl.ds(NPAD + base, _MROWS)], cv1)
        pltpu.sync_copy(p2_hbm, p2v)
        pltpu.sync_copy(p3_hbm, p3v)
        p2a = p2v[...]
        p3a = p3v[...]
        p2 = jnp.full((16,), p2a[li], jnp.float32)
        p3 = jnp.full((16,), p3a[li], jnp.float32)

        def gbody(g, carry):
            r0 = g * 16
            cw = cv0[pl.ds(r0, 16)] + cv1[pl.ds(r0, 16)]
            for k in range(16):
                r = r0 + k
                S = sv0[r, :] + sv1[r, :]
                cntv = jnp.full((16,), cw[k], jnp.float32)
                inv = 1.0 / jnp.maximum(cntv, 1.0)
                a = p2 * S * inv + p3 * cntv * inv * hv[r, :]
                e = jnp.exp(2.0 * a)
                hv[r, :] = 1.0 - 2.0 / (e + 1.0)
            return carry

        lax.fori_loop(0, _MROWS // 16, gbody, 0)
        pltpu.sync_copy(hv, hn_hbm.at[pl.ds(base, _MROWS)])

    return _merge


_sc_merge0 = _make_merge(0)
_sc_merge1 = _make_merge(1)


@functools.partial(
    pl.kernel,
    mesh=_SC_MESH,
    compiler_params=_SC_PARAMS,
    out_type=jax.ShapeDtypeStruct((NW, B), jnp.float32),
    scratch_types=[
        pltpu.VMEM((_MROWS, B), jnp.float32),
        pltpu.VMEM((_MROWS, B), jnp.float32),
        pltpu.VMEM((_MROWS, B), jnp.float32),
        pltpu.VMEM((_MROWS,), jnp.float32),
        pltpu.VMEM((_MROWS,), jnp.float32),
        pltpu.VMEM((_MROWS,), jnp.float32),
        pltpu.VMEM((16,), jnp.float32),
        pltpu.VMEM((16,), jnp.float32),
        pltpu.VMEM((16,), jnp.float32),
    ],
)
def _sc_merge_fin(s_hbm, c_hbm, h_hbm, p2_hbm, p3_hbm, w_hbm, d_out,
                  sv0, sv1, hv, cv0, cv1, wv, p2v, p3v, accv):
    c = lax.axis_index("c")
    s = lax.axis_index("s")
    wid = s * 2 + c
    base = wid * _MROWS
    pltpu.sync_copy(s_hbm.at[pl.ds(base, _MROWS)], sv0)
    pltpu.sync_copy(s_hbm.at[pl.ds(NPAD + base, _MROWS)], sv1)
    pltpu.sync_copy(h_hbm.at[pl.ds(base, _MROWS)], hv)
    pltpu.sync_copy(c_hbm.at[pl.ds(base, _MROWS)], cv0)
    pltpu.sync_copy(c_hbm.at[pl.ds(NPAD + base, _MROWS)], cv1)
    pltpu.sync_copy(w_hbm.at[pl.ds(base, _MROWS)], wv)
    pltpu.sync_copy(p2_hbm, p2v)
    pltpu.sync_copy(p3_hbm, p3v)
    p2a = p2v[...]
    p3a = p3v[...]
    p2 = jnp.full((16,), p2a[2], jnp.float32)
    p3 = jnp.full((16,), p3a[2], jnp.float32)

    def gbody(g, acc):
        r0 = g * 16
        cw = cv0[pl.ds(r0, 16)] + cv1[pl.ds(r0, 16)]
        ww = wv[pl.ds(r0, 16)]
        for k in range(16):
            r = r0 + k
            S = sv0[r, :] + sv1[r, :]
            cntv = jnp.full((16,), cw[k], jnp.float32)
            inv = 1.0 / jnp.maximum(cntv, 1.0)
            a = p2 * S * inv + p3 * cntv * inv * hv[r, :]
            e = jnp.exp(2.0 * a)
            h3 = 1.0 - 2.0 / (e + 1.0)
            acc = acc + h3 * jnp.full((16,), ww[k], jnp.float32)
        return acc

    acc = lax.fori_loop(0, _MROWS // 16, gbody,
                        jnp.zeros((16,), jnp.float32))
    accv[...] = acc
    pltpu.sync_copy(accv, d_out.at[wid])


def _fin_body(d_ref, b_ref, out_ref):
    out_ref[...] = jnp.sum(d_ref[...], axis=0).reshape(B, 1) + b_ref[0, 0]


def _fin_call(dots, b2):
    return pl.pallas_call(
        _fin_body,
        out_shape=jax.ShapeDtypeStruct((B, 1), jnp.float32),
    )(dots, b2)


def kernel(x, edge, edge_weight, param, p2s, p3s, W, b):
    srcp, dstp = _prep_call(edge)
    c1 = _sc_cnt(dstp)
    h0 = _h0_call(x, param)
    h = jnp.concatenate(
        [h0, jnp.zeros((NPAD - N, B), jnp.float32)], axis=0)

    zpad = jnp.zeros((16 - NGCN,), jnp.float32)
    ab2 = jnp.concatenate([jnp.abs(p2s).reshape(NGCN), zpad])
    ab3 = jnp.concatenate([jnp.abs(p3s).reshape(NGCN), zpad])
    wpad = jnp.concatenate([W[0], jnp.zeros((NPAD - N,), jnp.float32)])

    s1 = _sc_layer(h, srcp, dstp)
    h = _sc_merge0(s1, c1, h, ab2, ab3)
    s2 = _sc_layer(h, srcp, dstp)
    h = _sc_merge1(s2, c1, h, ab2, ab3)
    s3 = _sc_layer(h, srcp, dstp)
    dots = _sc_merge_fin(s3, c1, h, ab2, ab3, wpad)

    b2 = b.reshape(1, 1).astype(jnp.float32)
    return _fin_call(dots, b2)


# speedup vs baseline: 17.2181x; 1.0228x over previous
import functools

import jax
import jax.numpy as jnp
from jax import lax
from jax.experimental import pallas as pl
from jax.experimental.pallas import tpu as pltpu
from jax.experimental.pallas import tpu_sc as plsc

N = 10000
E = 160000
B = 16
D = 256
NGCN = 3

CH = 512
NCH0 = 15
NCH1 = 5
NPAIR = NCH0 + NCH1
NW = 32
EPAD = 16 * NPAIR * CH
NPAD = 10240
ROWS_PER_SUB = NPAD // 16
NBUF = 8

_EXTRA = EPAD - E


def _split_edges(row, fill):
    a = row.reshape(16, NPAIR * CH)
    ev = a[:, :NCH0 * CH].reshape(16, 1, NCH0, CH)
    od = jnp.concatenate(
        [a[:, NCH0 * CH:].reshape(16, NCH1, CH),
         fill.reshape(16, NCH0 - NCH1, CH)], axis=1).reshape(16, 1, NCH0, CH)
    return jnp.concatenate([ev, od], axis=1).reshape(NW, NCH0, CH)


def _prep_body(e_ref, src_ref, dst_ref):
    psrc = jnp.full((1, _EXTRA), N, jnp.int32)
    pdst = N + lax.rem(lax.broadcasted_iota(jnp.int32, (1, _EXTRA), 1),
                       NPAD - N)
    srcfull = jnp.concatenate([e_ref[0:1, :], psrc], axis=1)
    dstfull = jnp.concatenate([e_ref[1:2, :], pdst], axis=1)
    fill = jnp.full((16 * (NCH0 - NCH1) * CH,), N, jnp.int32)
    src_ref[...] = _split_edges(srcfull, fill)
    dst_ref[...] = _split_edges(dstfull, fill)


def _prep_call(edge):
    return pl.pallas_call(
        _prep_body,
        out_shape=(jax.ShapeDtypeStruct((NW, NCH0, CH), jnp.int32),
                   jax.ShapeDtypeStruct((NW, NCH0, CH), jnp.int32)),
    )(edge)


_NB = 1000


def _h0_body(x_ref, p_ref, o_ref):
    xb = x_ref[...]
    p = jnp.abs(p_ref[...])
    acc = lax.dot_general(xb.reshape(B * _NB, D), p,
                          (((1,), (0,)), ((), ())),
                          preferred_element_type=jnp.float32)
    h = jnp.tanh(acc) * (1.0 / D)
    o_ref[...] = jnp.transpose(h.reshape(B, _NB), (1, 0))


def _h0_call(x, param):
    return pl.pallas_call(
        _h0_body,
        grid=(N // _NB,),
        in_specs=[
            pl.BlockSpec((B, _NB, D), lambda i: (0, i, 0)),
            pl.BlockSpec((D, 1), lambda i: (0, 0)),
        ],
        out_specs=pl.BlockSpec((_NB, B), lambda i: (i, 0)),
        out_shape=jax.ShapeDtypeStruct((N, B), jnp.float32),
    )(x, param)


_SC_MESH = plsc.VectorSubcoreMesh(core_axis_name="c", subcore_axis_name="s")
_SC_PARAMS = pltpu.CompilerParams(use_tc_tiling_on_sc=False)

_GSEMS = [pltpu.SemaphoreType.DMA] * NBUF
_SSEMS = [pltpu.SemaphoreType.DMA] * NBUF


@functools.partial(
    pl.kernel,
    mesh=_SC_MESH,
    compiler_params=_SC_PARAMS,
    out_type=jax.ShapeDtypeStruct((2 * NPAD, B), jnp.float32),
    scratch_types=[
        pltpu.VMEM((NCH0, CH), jnp.int32),
        pltpu.VMEM((NCH0, CH), jnp.int32),
        pltpu.VMEM((NBUF, CH, B), jnp.float32),
        pltpu.VMEM((ROWS_PER_SUB, B), jnp.float32),
        pltpu.VMEM_SHARED((NPAD, B), jnp.float32),
    ] + _GSEMS + _SSEMS,
)
def _sc_layer(h_hbm, src_hbm, dst_hbm, s_out, srcv, dstv, gb, zbuf, s_sh,
              *sems):
    gsems = sems[:NBUF]
    ssems = sems[NBUF:]
    c = lax.axis_index("c")
    s = lax.axis_index("s")
    wid = s * 2 + c
    nch = NCH0 - (NCH0 - NCH1) * c
    pltpu.sync_copy(src_hbm.at[wid], srcv)
    pltpu.sync_copy(dst_hbm.at[wid], dstv)
    base = s * ROWS_PER_SUB

    for j in range(NBUF):
        @pl.when(j < nch)
        def _():
            pltpu.async_copy(h_hbm.at[srcv.at[j]], gb.at[j], gsems[j])

    zbuf[...] = jnp.zeros((ROWS_PER_SUB, B), jnp.float32)
    pltpu.sync_copy(zbuf, s_sh.at[pl.ds(base, ROWS_PER_SUB)])
    plsc.subcore_barrier()

    for j in range(NCH0):
        b = j % NBUF

        @pl.when(j < nch)
        def _():
            pltpu.make_async_copy(h_hbm.at[srcv.at[j]], gb.at[b],
                                  gsems[b]).wait()
            pltpu.async_copy(gb.at[b], s_sh.at[dstv.at[j]], ssems[b],
                             add=True)

        nj = j + NBUF
        if nj < NCH0:
            nb = nj % NBUF

            @pl.when(nj < nch)
            def _():
                pltpu.make_async_copy(gb.at[nb], s_sh.at[dstv.at[j]],
                                      ssems[nb]).wait()
                pltpu.async_copy(h_hbm.at[srcv.at[nj]], gb.at[nb], gsems[nb])

    for j in range(NCH0):
        b = j % NBUF

        @pl.when((j < nch) & (j >= nch - NBUF))
        def _():
            pltpu.make_async_copy(gb.at[b], s_sh.at[dstv.at[j]],
                                  ssems[b]).wait()
    plsc.subcore_barrier()
    pltpu.sync_copy(s_sh.at[pl.ds(base, ROWS_PER_SUB)],
                    s_out.at[pl.ds(c * NPAD + base, ROWS_PER_SUB)])


@functools.partial(
    pl.kernel,
    mesh=_SC_MESH,
    compiler_params=_SC_PARAMS,
    out_type=jax.ShapeDtypeStruct((2 * NPAD,), jnp.float32),
    scratch_types=[
        pltpu.VMEM((NCH0, CH), jnp.int32),
        pltpu.VMEM((CH,), jnp.float32),
        pltpu.VMEM((ROWS_PER_SUB,), jnp.float32),
        pltpu.VMEM_SHARED((NPAD,), jnp.float32),
        pltpu.SemaphoreType.DMA,
    ],
)
def _sc_cnt(dst_hbm, c_out, dstv, obuf, zbuf, c_sh, ssem):
    c = lax.axis_index("c")
    s = lax.axis_index("s")
    wid = s * 2 + c
    nch = NCH0 - (NCH0 - NCH1) * c
    pltpu.sync_copy(dst_hbm.at[wid], dstv)
    obuf[...] = jnp.full((CH,), 1.0, jnp.float32)
    base = s * ROWS_PER_SUB
    zbuf[...] = jnp.zeros((ROWS_PER_SUB,), jnp.float32)
    pltpu.sync_copy(zbuf, c_sh.at[pl.ds(base, ROWS_PER_SUB)])
    plsc.subcore_barrier()

    for j in range(NCH0):
        @pl.when(j < nch)
        def _():
            pltpu.async_copy(obuf, c_sh.at[dstv.at[j]], ssem, add=True)
    for j in range(NCH0):
        @pl.when(j < nch)
        def _():
            pltpu.make_async_copy(obuf, c_sh.at[dstv.at[j]], ssem).wait()
    plsc.subcore_barrier()
    pltpu.sync_copy(c_sh.at[pl.ds(base, ROWS_PER_SUB)],
                    c_out.at[pl.ds(c * NPAD + base, ROWS_PER_SUB)])


_MROWS = NPAD // NW


def _make_merge(li):
    @functools.partial(
        pl.kernel,
        mesh=_SC_MESH,
        compiler_params=_SC_PARAMS,
        out_type=jax.ShapeDtypeStruct((NPAD, B), jnp.float32),
        scratch_types=[
            pltpu.VMEM((_MROWS, B), jnp.float32),
            pltpu.VMEM((_MROWS, B), jnp.float32),
            pltpu.VMEM((_MROWS, B), jnp.float32),
            pltpu.VMEM((_MROWS,), jnp.float32),
            pltpu.VMEM((_MROWS,), jnp.float32),
            pltpu.VMEM((16,), jnp.float32),
            pltpu.VMEM((16,), jnp.float32),
        ],
    )
    def _merge(s_hbm, c_hbm, h_hbm, p2_hbm, p3_hbm, hn_hbm,
               sv0, sv1, hv, cv0, cv1, p2v, p3v):
        c = lax.axis_index("c")
        s = lax.axis_index("s")
        wid = s * 2 + c
        base = wid * _MROWS
        pltpu.sync_copy(s_hbm.at[pl.ds(base, _MROWS)], sv0)
        pltpu.sync_copy(s_hbm.at[pl.ds(NPAD + base, _MROWS)], sv1)
        pltpu.sync_copy(h_hbm.at[pl.ds(base, _MROWS)], hv)
        pltpu.sync_copy(c_hbm.at[pl.ds(base, _MROWS)], cv0)
        pltpu.sync_copy(c_hbm.at[pl.ds(NPAD + base, _MROWS)], cv1)
        pltpu.sync_copy(p2_hbm, p2v)
        pltpu.sync_copy(p3_hbm, p3v)
        p2a = p2v[...]
        p3a = p3v[...]
        p2 = jnp.full((16,), p2a[li], jnp.float32)
        p3 = jnp.full((16,), p3a[li], jnp.float32)

        def gbody(g, carry):
            r0 = g * 16
            cw = cv0[pl.ds(r0, 16)] + cv1[pl.ds(r0, 16)]
            for k in range(16):
                r = r0 + k
                S = sv0[r, :] + sv1[r, :]
                cntv = jnp.full((16,), cw[k], jnp.float32)
                inv = 1.0 / jnp.maximum(cntv, 1.0)
                a = p2 * S * inv + p3 * cntv * inv * hv[r, :]
                e = jnp.exp(2.0 * a)
                hv[r, :] = 1.0 - 2.0 / (e + 1.0)
            return carry

        lax.fori_loop(0, _MROWS // 16, gbody, 0)
        pltpu.sync_copy(hv, hn_hbm.at[pl.ds(base, _MROWS)])

    return _merge


_sc_merge0 = _make_merge(0)
_sc_merge1 = _make_merge(1)


@functools.partial(
    pl.kernel,
    mesh=_SC_MESH,
    compiler_params=_SC_PARAMS,
    out_type=jax.ShapeDtypeStruct((NW, B), jnp.float32),
    scratch_types=[
        pltpu.VMEM((_MROWS, B), jnp.float32),
        pltpu.VMEM((_MROWS, B), jnp.float32),
        pltpu.VMEM((_MROWS, B), jnp.float32),
        pltpu.VMEM((_MROWS,), jnp.float32),
        pltpu.VMEM((_MROWS,), jnp.float32),
        pltpu.VMEM((_MROWS,), jnp.float32),
        pltpu.VMEM((16,), jnp.float32),
        pltpu.VMEM((16,), jnp.float32),
        pltpu.VMEM((16,), jnp.float32),
    ],
)
def _sc_merge_fin(s_hbm, c_hbm, h_hbm, p2_hbm, p3_hbm, w_hbm, d_out,
                  sv0, sv1, hv, cv0, cv1, wv, p2v, p3v, accv):
    c = lax.axis_index("c")
    s = lax.axis_index("s")
    wid = s * 2 + c
    base = wid * _MROWS
    pltpu.sync_copy(s_hbm.at[pl.ds(base, _MROWS)], sv0)
    pltpu.sync_copy(s_hbm.at[pl.ds(NPAD + base, _MROWS)], sv1)
    pltpu.sync_copy(h_hbm.at[pl.ds(base, _MROWS)], hv)
    pltpu.sync_copy(c_hbm.at[pl.ds(base, _MROWS)], cv0)
    pltpu.sync_copy(c_hbm.at[pl.ds(NPAD + base, _MROWS)], cv1)
    pltpu.sync_copy(w_hbm.at[pl.ds(base, _MROWS)], wv)
    pltpu.sync_copy(p2_hbm, p2v)
    pltpu.sync_copy(p3_hbm, p3v)
    p2a = p2v[...]
    p3a = p3v[...]
    p2 = jnp.full((16,), p2a[2], jnp.float32)
    p3 = jnp.full((16,), p3a[2], jnp.float32)

    def gbody(g, acc):
        r0 = g * 16
        cw = cv0[pl.ds(r0, 16)] + cv1[pl.ds(r0, 16)]
        ww = wv[pl.ds(r0, 16)]
        for k in range(16):
            r = r0 + k
            S = sv0[r, :] + sv1[r, :]
            cntv = jnp.full((16,), cw[k], jnp.float32)
            inv = 1.0 / jnp.maximum(cntv, 1.0)
            a = p2 * S * inv + p3 * cntv * inv * hv[r, :]
            e = jnp.exp(2.0 * a)
            h3 = 1.0 - 2.0 / (e + 1.0)
            acc = acc + h3 * jnp.full((16,), ww[k], jnp.float32)
        return acc

    acc = lax.fori_loop(0, _MROWS // 16, gbody,
                        jnp.zeros((16,), jnp.float32))
    accv[...] = acc
    pltpu.sync_copy(accv, d_out.at[wid])


def _fin_body(d_ref, b_ref, out_ref):
    out_ref[...] = jnp.sum(d_ref[...], axis=0).reshape(B, 1) + b_ref[0, 0]


def _fin_call(dots, b2):
    return pl.pallas_call(
        _fin_body,
        out_shape=jax.ShapeDtypeStruct((B, 1), jnp.float32),
    )(dots, b2)


def kernel(x, edge, edge_weight, param, p2s, p3s, W, b):
    srcp, dstp = _prep_call(edge)
    c1 = _sc_cnt(dstp)
    h0 = _h0_call(x, param)
    h = jnp.concatenate(
        [h0, jnp.zeros((NPAD - N, B), jnp.float32)], axis=0)

    zpad = jnp.zeros((16 - NGCN,), jnp.float32)
    ab2 = jnp.concatenate([jnp.abs(p2s).reshape(NGCN), zpad])
    ab3 = jnp.concatenate([jnp.abs(p3s).reshape(NGCN), zpad])
    wpad = jnp.concatenate([W[0], jnp.zeros((NPAD - N,), jnp.float32)])

    s1 = _sc_layer(h, srcp, dstp)
    h = _sc_merge0(s1, c1, h, ab2, ab3)
    s2 = _sc_layer(h, srcp, dstp)
    h = _sc_merge1(s2, c1, h, ab2, ab3)
    s3 = _sc_layer(h, srcp, dstp)
    dots = _sc_merge_fin(s3, c1, h, ab2, ab3, wpad)

    b2 = b.reshape(1, 1).astype(jnp.float32)
    return _fin_call(dots, b2)
